# baseline (speedup 1.0000x reference)
import jax
import jax.numpy as jnp
from jax import lax
from jax.experimental import pallas as pl
from jax.experimental.pallas import tpu as pltpu
from jax.experimental.pallas import tpu_sc as plsc

T = 2048
D = 2048
E = 8
F = 768
K = 2

BM = 128
NB = 40
NP = NB * BM

NC, NS = 2, 16
NW = NC * NS
_DH = D // 2
_TPW = T // NW


_RB = 512
_NBL = T // _RB


def _router_body(x_ref, gw_ref, w1_ref, w2_ref, xb_ref, pe_ref, po_ref,
                 be_ref, xbi_ref, act_ref, cnt_ref):
    b = pl.program_id(0)
    x = x_ref[...]
    gw = gw_ref[...]
    logits = lax.dot_general(x, gw, (((1,), (1,)), ((), ())),
                             preferred_element_type=jnp.float32)
    iota = lax.broadcasted_iota(jnp.int32, logits.shape, 1)
    m1 = jnp.max(logits, axis=1, keepdims=True)
    i1 = jnp.min(jnp.where(logits == m1, iota, E), axis=1)
    oh1 = (iota == i1[:, None]).astype(jnp.float32)
    masked = jnp.where(oh1 > 0, -jnp.inf, logits)
    m2 = jnp.max(masked, axis=1, keepdims=True)
    i2 = jnp.min(jnp.where(masked == m2, iota, E), axis=1)
    oh2 = (iota == i2[:, None]).astype(jnp.float32)

    @pl.when(b < _NBL)
    def _pass1():
        w1 = jax.nn.sigmoid(m1 - m2)
        w1_ref[...] = jnp.broadcast_to(w1, (_RB, 16))
        w2_ref[...] = jnp.broadcast_to(1.0 - w1, (_RB, 16))

        def _bf16_bits(v):
            u = lax.bitcast_convert_type(v, jnp.int32)
            return (u + 0x7FFF + ((u >> 16) & 1)) >> 16

        blo = _bf16_bits(x[:, :D // 2]) & 0xFFFF
        bhi = _bf16_bits(x[:, D // 2:])
        xb_ref[...] = blo | (bhi << 16)
        cnt_ref[pl.ds(b, 1), :] = jnp.sum(oh1 + oh2, axis=0,
                                          keepdims=True)

    @pl.when(b >= _NBL)
    def _pass2():
        bb = b - _NBL
        rows = cnt_ref[...]
        r_iota = lax.broadcasted_iota(jnp.int32, rows.shape, 0)
        c_base = jnp.sum(jnp.where(r_iota < bb, rows, 0.0),
                         axis=0, keepdims=True)
        totals = jnp.sum(rows, axis=0, keepdims=True)
        tot_i = totals.astype(jnp.int32)
        padded = ((tot_i + BM - 1) // BM) * BM
        padded_f = padded.astype(jnp.float32)
        le_i = lax.broadcasted_iota(jnp.int32, (E, E), 0)
        le_j = lax.broadcasted_iota(jnp.int32, (E, E), 1)
        ltri8 = (le_i < le_j).astype(jnp.float32)
        pad_off = lax.dot_general(padded_f, ltri8,
                                  (((1,), (0,)), ((), ())),
                                  preferred_element_type=jnp.float32)
        tt_i = lax.broadcasted_iota(jnp.int32, (_RB, _RB), 0)
        tt_j = lax.broadcasted_iota(jnp.int32, (_RB, _RB), 1)
        strict = (tt_i > tt_j).astype(jnp.float32)
        p_strict = lax.dot_general(strict, oh1 + oh2,
                                   (((1,), (0,)), ((), ())),
                                   preferred_element_type=jnp.float32)
        m = pad_off + c_base + p_strict
        dest1 = jnp.sum(oh1 * m, axis=1).astype(jnp.int32)
        dest2 = jnp.sum(oh2 * m, axis=1).astype(jnp.int32)
        pe_ref[...] = dest1.reshape(_RB // _TPW, _TPW)
        po_ref[...] = dest2.reshape(_RB // _TPW, _TPW)

        @pl.when(b == 2 * _NBL - 1)
        def _tables():
            pad_end = pad_off + padded_f
            nb = (jnp.sum(padded_f) / BM).astype(jnp.int32)
            bi2 = lax.broadcasted_iota(jnp.int32, (NB, E), 0)
            be_raw = jnp.sum((bi2.astype(jnp.float32) * BM >=
                              pad_end).astype(jnp.int32), axis=1)
            b1 = lax.broadcasted_iota(jnp.int32, (NB,), 0)
            active = b1 < nb
            e_last = jnp.sum(jnp.where(b1 == nb - 1, be_raw, 0))
            be_ref[...] = jnp.where(active, be_raw, e_last).astype(jnp.int32)
            xbi_ref[...] = jnp.where(active, b1, nb - 1).astype(jnp.int32)
            act_ref[...] = active.astype(jnp.int32)


def _router_meta(x, gate_weight):
    wpb = _RB // _TPW
    return pl.pallas_call(
        _router_body,
        grid=(2 * _NBL,),
        in_specs=[
            pl.BlockSpec((_RB, D), lambda b: (b % _NBL, 0)),
            pl.BlockSpec((E, D), lambda b: (0, 0)),
        ],
        out_specs=[
            pl.BlockSpec((_RB, 16), lambda b: (jnp.minimum(b, _NBL - 1), 0)),
            pl.BlockSpec((_RB, 16), lambda b: (jnp.minimum(b, _NBL - 1), 0)),
            pl.BlockSpec((_RB, _DH),
                         lambda b: (jnp.minimum(b, _NBL - 1), 0)),
            pl.BlockSpec((wpb, _TPW),
                         lambda b: (jnp.maximum(b - _NBL, 0), 0)),
            pl.BlockSpec((wpb, _TPW),
                         lambda b: (jnp.maximum(b - _NBL, 0), 0)),
            pl.BlockSpec((NB,), lambda b: (0,)),
            pl.BlockSpec((NB,), lambda b: (0,)),
            pl.BlockSpec((NB,), lambda b: (0,)),
        ],
        out_shape=[
            jax.ShapeDtypeStruct((T, 16), jnp.float32),
            jax.ShapeDtypeStruct((T, 16), jnp.float32),
            jax.ShapeDtypeStruct((T, _DH), jnp.int32),
            jax.ShapeDtypeStruct((NW, _TPW), jnp.int32),
            jax.ShapeDtypeStruct((NW, _TPW), jnp.int32),
            jax.ShapeDtypeStruct((NB,), jnp.int32),
            jax.ShapeDtypeStruct((NB,), jnp.int32),
            jax.ShapeDtypeStruct((NB,), jnp.int32),
        ],
        scratch_shapes=[pltpu.VMEM((_NBL, E), jnp.float32)],
        compiler_params=pltpu.CompilerParams(
            dimension_semantics=("arbitrary",)),
    )(x, gate_weight)


def _scatter_body(xb_hbm, pe_hbm, po_hbm, xs_hbm,
                  ie_v, io_v, rows_v, s1, s2):
    wid = lax.axis_index("s") * NC + lax.axis_index("c")
    tb = wid * _TPW
    pltpu.sync_copy(pe_hbm.at[wid], ie_v)
    pltpu.sync_copy(po_hbm.at[wid], io_v)
    pltpu.sync_copy(xb_hbm.at[pl.ds(tb, _TPW)], rows_v)
    c1 = pltpu.async_copy(rows_v, xs_hbm.at[ie_v], s1)
    c2 = pltpu.async_copy(rows_v, xs_hbm.at[io_v], s2)
    c1.wait()
    c2.wait()


def _dispatch_scatter(xb, pe, po):
    mesh = plsc.VectorSubcoreMesh(core_axis_name="c", subcore_axis_name="s")
    run = pl.kernel(
        _scatter_body,
        out_type=jax.ShapeDtypeStruct((NP, _DH), jnp.int32),
        mesh=mesh,
        scratch_types=[
            pltpu.VMEM((_TPW,), jnp.int32),
            pltpu.VMEM((_TPW,), jnp.int32),
            pltpu.VMEM((_TPW, _DH), jnp.int32),
            pltpu.SemaphoreType.DMA,
            pltpu.SemaphoreType.DMA,
        ],
    )
    return run(xb, pe, po)


def _ffn_body(be_ref, xbi_ref, act_ref, xs_ref,
              wg_ref, wu_ref, wd_ref, ys_ref):
    b = pl.program_id(0)

    @pl.when(act_ref[b] == 1)
    def _():
        packed = xs_ref[...]
        xlo = lax.bitcast_convert_type(packed << 16, jnp.float32)
        xhi = lax.bitcast_convert_type(packed & jnp.int32(-65536),
                                       jnp.float32)
        xb = jnp.concatenate([xlo, xhi], axis=1).astype(jnp.bfloat16)
        wg = wg_ref[0].astype(jnp.bfloat16)
        wu = wu_ref[0].astype(jnp.bfloat16)
        wd = wd_ref[0].astype(jnp.bfloat16)
        g = jnp.dot(xb, wg, preferred_element_type=jnp.float32)
        u = jnp.dot(xb, wu, preferred_element_type=jnp.float32)
        h = (g * jax.nn.sigmoid(g)) * u
        y = jnp.dot(h.astype(jnp.bfloat16), wd,
                    preferred_element_type=jnp.float32)

        def _bf16_bits(v):
            u32 = lax.bitcast_convert_type(v, jnp.int32)
            return (u32 + 0x7FFF + ((u32 >> 16) & 1)) >> 16

        blo = _bf16_bits(y[:, :D // 2]) & 0xFFFF
        bhi = _bf16_bits(y[:, D // 2:])
        ys_ref[...] = lax.bitcast_convert_type(blo | (bhi << 16),
                                               jnp.float32)


def _grouped_ffn(xs, w_gate, w_up, w_down, be, xbi, act):
    grid_spec = pltpu.PrefetchScalarGridSpec(
        num_scalar_prefetch=3,
        grid=(NB,),
        in_specs=[
            pl.BlockSpec((BM, _DH), lambda b, be, xbi, act: (xbi[b], 0)),
            pl.BlockSpec((1, D, F), lambda b, be, xbi, act: (be[b], 0, 0)),
            pl.BlockSpec((1, D, F), lambda b, be, xbi, act: (be[b], 0, 0)),
            pl.BlockSpec((1, F, D), lambda b, be, xbi, act: (be[b], 0, 0)),
        ],
        out_specs=pl.BlockSpec((BM, _DH), lambda b, be, xbi, act: (xbi[b], 0)),
    )
    return pl.pallas_call(
        _ffn_body,
        grid_spec=grid_spec,
        out_shape=jax.ShapeDtypeStruct((NP, _DH), jnp.float32),
        compiler_params=pltpu.CompilerParams(
            dimension_semantics=("arbitrary",)),
    )(be, xbi, act, xs, w_gate, w_up, w_down)


_CCH = 32
_CNC = _TPW // _CCH


def _combine_body(ys_hbm, pe_hbm, po_hbm, w1_hbm, w2_hbm, out_hbm,
                  i1_v, i2_v, w1_v, w2_v, r1, r2, sg1, sg2, sw1, sw2):
    wid = lax.axis_index("s") * NC + lax.axis_index("c")
    base = wid * _TPW
    pltpu.sync_copy(pe_hbm.at[wid], i1_v)
    pltpu.sync_copy(po_hbm.at[wid], i2_v)
    pltpu.sync_copy(w1_hbm.at[pl.ds(base, _TPW)], w1_v)
    pltpu.sync_copy(w2_hbm.at[pl.ds(base, _TPW)], w2_v)

    mhi = jnp.full((16,), -65536, jnp.int32)
    wb1 = wb2 = None
    for c in range(_CNC):
        if wb1 is not None:
            wb1.wait()
            wb2.wait()
        sl_idx = pl.ds(c * _CCH, _CCH)
        g1 = pltpu.async_copy(ys_hbm.at[i1_v.at[sl_idx]], r1, sg1)
        g2 = pltpu.async_copy(ys_hbm.at[i2_v.at[sl_idx]], r2, sg2)
        g1.wait()
        g2.wait()

        def add_row(r, _):
            v1 = w1_v[c * _CCH + r, pl.ds(0, 16)]
            v2 = w2_v[c * _CCH + r, pl.ds(0, 16)]
            bc = lax.bitcast_convert_type

            def add_vec(jb, _):
                for ju in range(8):
                    sl = pl.ds(jb * 128 + ju * 16, 16)
                    p1v = bc(r1[r, sl], jnp.int32)
                    p2v = bc(r2[r, sl], jnp.int32)
                    lo = (bc(p1v << 16, jnp.float32) * v1
                          + bc(p2v << 16, jnp.float32) * v2)
                    hi = (bc(p1v & mhi, jnp.float32) * v1
                          + bc(p2v & mhi, jnp.float32) * v2)
                    r1[r, sl] = lo
                    r2[r, sl] = hi
                return 0
            lax.fori_loop(0, _DH // 128, add_vec, 0)
            return 0

        lax.fori_loop(0, _CCH, add_row, 0)
        out_rows = pl.ds(base + c * _CCH, _CCH)
        wb1 = pltpu.async_copy(r1, out_hbm.at[out_rows, pl.ds(0, _DH)], sw1)
        wb2 = pltpu.async_copy(r2, out_hbm.at[out_rows, pl.ds(_DH, _DH)],
                               sw2)
    wb1.wait()
    wb2.wait()


def _combine(ys, pe, po, w1, w2):
    mesh = plsc.VectorSubcoreMesh(core_axis_name="c", subcore_axis_name="s")
    run = pl.kernel(
        _combine_body,
        out_type=jax.ShapeDtypeStruct((T, D), jnp.float32),
        mesh=mesh,
        scratch_types=(
            [pltpu.VMEM((_TPW,), jnp.int32)] * 2
            + [pltpu.VMEM((_TPW, 16), jnp.float32)] * 2
            + [pltpu.VMEM((_CCH, _DH), jnp.float32)] * 2
            + [pltpu.SemaphoreType.DMA] * 4
        ),
    )
    return run(ys, pe, po, w1, w2)


def kernel(hidden_states, gate_weight, w_gate_proj, w_up_proj, w_down_proj):
    x = hidden_states.reshape(T, D)
    w1, w2, xb, pe, po, be, xbi, act = _router_meta(x, gate_weight)
    xs = _dispatch_scatter(xb, pe, po)
    ys = _grouped_ffn(xs, w_gate_proj, w_up_proj, w_down_proj, be, xbi, act)
    out = _combine(ys, pe, po, w1, w2)
    return out.reshape(hidden_states.shape)


# speedup vs baseline: 1.0267x; 1.0267x over previous
import jax
import jax.numpy as jnp
from jax import lax
from jax.experimental import pallas as pl
from jax.experimental.pallas import tpu as pltpu
from jax.experimental.pallas import tpu_sc as plsc

T = 2048
D = 2048
E = 8
F = 768
K = 2

BM = 128
NB = 40
NP = NB * BM

NC, NS = 2, 16
NW = NC * NS
_DH = D // 2
_TPW = T // NW


_RB = 512
_NBL = T // _RB


def _router_body(x_ref, gw_ref, w1_ref, w2_ref, xb_ref, pe_ref, po_ref,
                 be_ref, xbi_ref, act_ref, cnt_ref):
    b = pl.program_id(0)
    x = x_ref[...]
    gw = gw_ref[...]
    logits = lax.dot_general(x, gw, (((1,), (1,)), ((), ())),
                             preferred_element_type=jnp.float32)
    iota = lax.broadcasted_iota(jnp.int32, logits.shape, 1)
    m1 = jnp.max(logits, axis=1, keepdims=True)
    i1 = jnp.min(jnp.where(logits == m1, iota, E), axis=1)
    oh1 = (iota == i1[:, None]).astype(jnp.float32)
    masked = jnp.where(oh1 > 0, -jnp.inf, logits)
    m2 = jnp.max(masked, axis=1, keepdims=True)
    i2 = jnp.min(jnp.where(masked == m2, iota, E), axis=1)
    oh2 = (iota == i2[:, None]).astype(jnp.float32)

    @pl.when(b < _NBL)
    def _pass1():
        w1 = jax.nn.sigmoid(m1 - m2)
        w1_ref[...] = jnp.broadcast_to(w1, (_RB, 16))
        w2_ref[...] = jnp.broadcast_to(1.0 - w1, (_RB, 16))

        def _bf16_bits(v):
            u = lax.bitcast_convert_type(v, jnp.int32)
            return (u + 0x7FFF + ((u >> 16) & 1)) >> 16

        blo = _bf16_bits(x[:, :D // 2]) & 0xFFFF
        bhi = _bf16_bits(x[:, D // 2:])
        xb_ref[...] = blo | (bhi << 16)
        cnt_ref[pl.ds(b, 1), :] = jnp.sum(oh1 + oh2, axis=0,
                                          keepdims=True)

    @pl.when(b >= _NBL)
    def _pass2():
        bb = b - _NBL
        rows = cnt_ref[...]
        r_iota = lax.broadcasted_iota(jnp.int32, rows.shape, 0)
        c_base = jnp.sum(jnp.where(r_iota < bb, rows, 0.0),
                         axis=0, keepdims=True)
        totals = jnp.sum(rows, axis=0, keepdims=True)
        tot_i = totals.astype(jnp.int32)
        padded = ((tot_i + BM - 1) // BM) * BM
        padded_f = padded.astype(jnp.float32)
        le_i = lax.broadcasted_iota(jnp.int32, (E, E), 0)
        le_j = lax.broadcasted_iota(jnp.int32, (E, E), 1)
        ltri8 = (le_i < le_j).astype(jnp.float32)
        pad_off = lax.dot_general(padded_f, ltri8,
                                  (((1,), (0,)), ((), ())),
                                  preferred_element_type=jnp.float32)
        tt_i = lax.broadcasted_iota(jnp.int32, (_RB, _RB), 0)
        tt_j = lax.broadcasted_iota(jnp.int32, (_RB, _RB), 1)
        strict = (tt_i > tt_j).astype(jnp.float32)
        p_strict = lax.dot_general(strict, oh1 + oh2,
                                   (((1,), (0,)), ((), ())),
                                   preferred_element_type=jnp.float32)
        m = pad_off + c_base + p_strict
        dest1 = jnp.sum(oh1 * m, axis=1).astype(jnp.int32)
        dest2 = jnp.sum(oh2 * m, axis=1).astype(jnp.int32)
        pe_ref[...] = dest1.reshape(_RB // _TPW, _TPW)
        po_ref[...] = dest2.reshape(_RB // _TPW, _TPW)

        @pl.when(b == 2 * _NBL - 1)
        def _tables():
            pad_end = pad_off + padded_f
            nb = (jnp.sum(padded_f) / BM).astype(jnp.int32)
            bi2 = lax.broadcasted_iota(jnp.int32, (NB, E), 0)
            be_raw = jnp.sum((bi2.astype(jnp.float32) * BM >=
                              pad_end).astype(jnp.int32), axis=1)
            b1 = lax.broadcasted_iota(jnp.int32, (NB,), 0)
            active = b1 < nb
            e_last = jnp.sum(jnp.where(b1 == nb - 1, be_raw, 0))
            be_ref[...] = jnp.where(active, be_raw, e_last).astype(jnp.int32)
            xbi_ref[...] = jnp.where(active, b1, nb - 1).astype(jnp.int32)
            act_ref[...] = active.astype(jnp.int32)


def _router_meta(x, gate_weight):
    wpb = _RB // _TPW
    return pl.pallas_call(
        _router_body,
        grid=(2 * _NBL,),
        in_specs=[
            pl.BlockSpec((_RB, D), lambda b: (b % _NBL, 0)),
            pl.BlockSpec((E, D), lambda b: (0, 0)),
        ],
        out_specs=[
            pl.BlockSpec((_RB, 16), lambda b: (jnp.minimum(b, _NBL - 1), 0)),
            pl.BlockSpec((_RB, 16), lambda b: (jnp.minimum(b, _NBL - 1), 0)),
            pl.BlockSpec((_RB, _DH),
                         lambda b: (jnp.minimum(b, _NBL - 1), 0)),
            pl.BlockSpec((wpb, _TPW),
                         lambda b: (jnp.maximum(b - _NBL, 0), 0)),
            pl.BlockSpec((wpb, _TPW),
                         lambda b: (jnp.maximum(b - _NBL, 0), 0)),
            pl.BlockSpec((NB,), lambda b: (0,)),
            pl.BlockSpec((NB,), lambda b: (0,)),
            pl.BlockSpec((NB,), lambda b: (0,)),
        ],
        out_shape=[
            jax.ShapeDtypeStruct((T, 16), jnp.float32),
            jax.ShapeDtypeStruct((T, 16), jnp.float32),
            jax.ShapeDtypeStruct((T, _DH), jnp.int32),
            jax.ShapeDtypeStruct((NW, _TPW), jnp.int32),
            jax.ShapeDtypeStruct((NW, _TPW), jnp.int32),
            jax.ShapeDtypeStruct((NB,), jnp.int32),
            jax.ShapeDtypeStruct((NB,), jnp.int32),
            jax.ShapeDtypeStruct((NB,), jnp.int32),
        ],
        scratch_shapes=[pltpu.VMEM((_NBL, E), jnp.float32)],
        compiler_params=pltpu.CompilerParams(
            dimension_semantics=("arbitrary",)),
    )(x, gate_weight)


def _scatter_body(xb_hbm, pe_hbm, po_hbm, xs_hbm,
                  ie_v, io_v, rows_v, s1, s2):
    wid = lax.axis_index("s") * NC + lax.axis_index("c")
    tb = wid * _TPW
    pltpu.sync_copy(pe_hbm.at[wid], ie_v)
    pltpu.sync_copy(po_hbm.at[wid], io_v)
    pltpu.sync_copy(xb_hbm.at[pl.ds(tb, _TPW)], rows_v)
    c1 = pltpu.async_copy(rows_v, xs_hbm.at[ie_v], s1)
    c2 = pltpu.async_copy(rows_v, xs_hbm.at[io_v], s2)
    c1.wait()
    c2.wait()


def _dispatch_scatter(xb, pe, po):
    mesh = plsc.VectorSubcoreMesh(core_axis_name="c", subcore_axis_name="s")
    run = pl.kernel(
        _scatter_body,
        out_type=jax.ShapeDtypeStruct((NP, _DH), jnp.int32),
        mesh=mesh,
        scratch_types=[
            pltpu.VMEM((_TPW,), jnp.int32),
            pltpu.VMEM((_TPW,), jnp.int32),
            pltpu.VMEM((_TPW, _DH), jnp.int32),
            pltpu.SemaphoreType.DMA,
            pltpu.SemaphoreType.DMA,
        ],
    )
    return run(xb, pe, po)


def _ffn_body(be_ref, xbi_ref, act_ref, xs_ref,
              wg_ref, wu_ref, wd_ref, ys_ref):
    b = pl.program_id(0)

    @pl.when(act_ref[b] == 1)
    def _():
        packed = xs_ref[...]
        xlo = lax.bitcast_convert_type(packed << 16, jnp.float32)
        xhi = lax.bitcast_convert_type(packed & jnp.int32(-65536),
                                       jnp.float32)
        xb = jnp.concatenate([xlo, xhi], axis=1).astype(jnp.bfloat16)
        wg = wg_ref[0].astype(jnp.bfloat16)
        wu = wu_ref[0].astype(jnp.bfloat16)
        wd = wd_ref[0].astype(jnp.bfloat16)
        g = jnp.dot(xb, wg, preferred_element_type=jnp.float32)
        u = jnp.dot(xb, wu, preferred_element_type=jnp.float32)
        h = (g * jax.nn.sigmoid(g)) * u
        y = jnp.dot(h.astype(jnp.bfloat16), wd,
                    preferred_element_type=jnp.float32)

        def _bf16_bits(v):
            u32 = lax.bitcast_convert_type(v, jnp.int32)
            return (u32 + 0x7FFF + ((u32 >> 16) & 1)) >> 16

        blo = _bf16_bits(y[:, :D // 2]) & 0xFFFF
        bhi = _bf16_bits(y[:, D // 2:])
        ys_ref[...] = lax.bitcast_convert_type(blo | (bhi << 16),
                                               jnp.float32)


def _grouped_ffn(xs, w_gate, w_up, w_down, be, xbi, act):
    grid_spec = pltpu.PrefetchScalarGridSpec(
        num_scalar_prefetch=3,
        grid=(NB,),
        in_specs=[
            pl.BlockSpec((BM, _DH), lambda b, be, xbi, act: (xbi[b], 0)),
            pl.BlockSpec((1, D, F), lambda b, be, xbi, act: (be[b], 0, 0)),
            pl.BlockSpec((1, D, F), lambda b, be, xbi, act: (be[b], 0, 0)),
            pl.BlockSpec((1, F, D), lambda b, be, xbi, act: (be[b], 0, 0)),
        ],
        out_specs=pl.BlockSpec((BM, _DH), lambda b, be, xbi, act: (xbi[b], 0)),
    )
    return pl.pallas_call(
        _ffn_body,
        grid_spec=grid_spec,
        out_shape=jax.ShapeDtypeStruct((NP, _DH), jnp.float32),
        compiler_params=pltpu.CompilerParams(
            dimension_semantics=("arbitrary",)),
    )(be, xbi, act, xs, w_gate, w_up, w_down)


_CCH = 16
_CNC = _TPW // _CCH
_CNB = 3


def _combine_body(ys_hbm, pe_hbm, po_hbm, w1_hbm, w2_hbm, out_hbm,
                  i1_v, i2_v, w1_v, w2_v, *scr):
    r1 = scr[:_CNB]
    r2 = scr[_CNB:2 * _CNB]
    sg1 = scr[2 * _CNB:3 * _CNB]
    sg2 = scr[3 * _CNB:4 * _CNB]
    sw1 = scr[4 * _CNB:5 * _CNB]
    sw2 = scr[5 * _CNB:]
    wid = lax.axis_index("s") * NC + lax.axis_index("c")
    base = wid * _TPW
    pltpu.sync_copy(pe_hbm.at[wid], i1_v)
    pltpu.sync_copy(po_hbm.at[wid], i2_v)
    pltpu.sync_copy(w1_hbm.at[pl.ds(base, _TPW)], w1_v)
    pltpu.sync_copy(w2_hbm.at[pl.ds(base, _TPW)], w2_v)

    def fire(c, s):
        sl = pl.ds(c * _CCH, _CCH)
        return (pltpu.async_copy(ys_hbm.at[i1_v.at[sl]], r1[s], sg1[s]),
                pltpu.async_copy(ys_hbm.at[i2_v.at[sl]], r2[s], sg2[s]))

    gd = [None] * _CNB
    wb = [None] * _CNB
    for c in range(min(_CNB - 1, _CNC)):
        gd[c] = fire(c, c)
    for c in range(_CNC):
        s = c % _CNB
        n = c + _CNB - 1
        if n < _CNC:
            sn = n % _CNB
            if wb[sn] is not None:
                wb[sn][0].wait()
                wb[sn][1].wait()
            gd[sn] = fire(n, sn)
        gd[s][0].wait()
        gd[s][1].wait()

        def add_row(r, _):
            v1 = w1_v[c * _CCH + r, pl.ds(0, 16)]
            v2 = w2_v[c * _CCH + r, pl.ds(0, 16)]
            bc = lax.bitcast_convert_type

            def add_vec(jb, _):
                for ju in range(8):
                    sl = pl.ds(jb * 128 + ju * 16, 16)
                    p1v = bc(r1[s][r, sl], jnp.int32)
                    p2v = bc(r2[s][r, sl], jnp.int32)
                    lo = (bc(p1v << 16, jnp.float32) * v1
                          + bc(p2v << 16, jnp.float32) * v2)
                    hi = (bc(p1v, jnp.float32) * v1
                          + bc(p2v, jnp.float32) * v2)
                    r1[s][r, sl] = lo
                    r2[s][r, sl] = hi
                return 0
            lax.fori_loop(0, _DH // 128, add_vec, 0)
            return 0

        lax.fori_loop(0, _CCH, add_row, 0)
        out_rows = pl.ds(base + c * _CCH, _CCH)
        wb[s] = (
            pltpu.async_copy(r1[s], out_hbm.at[out_rows, pl.ds(0, _DH)],
                             sw1[s]),
            pltpu.async_copy(r2[s], out_hbm.at[out_rows, pl.ds(_DH, _DH)],
                             sw2[s]))
    for s in range(_CNB):
        if wb[s] is not None:
            wb[s][0].wait()
            wb[s][1].wait()


def _combine(ys, pe, po, w1, w2):
    mesh = plsc.VectorSubcoreMesh(core_axis_name="c", subcore_axis_name="s")
    run = pl.kernel(
        _combine_body,
        out_type=jax.ShapeDtypeStruct((T, D), jnp.float32),
        mesh=mesh,
        scratch_types=(
            [pltpu.VMEM((_TPW,), jnp.int32)] * 2
            + [pltpu.VMEM((_TPW, 16), jnp.float32)] * 2
            + [pltpu.VMEM((_CCH, _DH), jnp.float32)] * (2 * _CNB)
            + [pltpu.SemaphoreType.DMA] * (4 * _CNB)
        ),
    )
    return run(ys, pe, po, w1, w2)


def kernel(hidden_states, gate_weight, w_gate_proj, w_up_proj, w_down_proj):
    x = hidden_states.reshape(T, D)
    w1, w2, xb, pe, po, be, xbi, act = _router_meta(x, gate_weight)
    xs = _dispatch_scatter(xb, pe, po)
    ys = _grouped_ffn(xs, w_gate_proj, w_up_proj, w_down_proj, be, xbi, act)
    out = _combine(ys, pe, po, w1, w2)
    return out.reshape(hidden_states.shape)


# speedup vs baseline: 1.0716x; 1.0438x over previous
import jax
import jax.numpy as jnp
from jax import lax
from jax.experimental import pallas as pl
from jax.experimental.pallas import tpu as pltpu
from jax.experimental.pallas import tpu_sc as plsc

T = 2048
D = 2048
E = 8
F = 768
K = 2

BM = 256
NB = 24
NP = NB * BM

NC, NS = 2, 16
NW = NC * NS
_DH = D // 2
_TPW = T // NW


_RB = 512
_NBL = T // _RB


def _router_body(x_ref, gw_ref, w1_ref, w2_ref, xb_ref, pe_ref, po_ref,
                 be_ref, xbi_ref, act_ref, cnt_ref):
    b = pl.program_id(0)
    x = x_ref[...]
    gw = gw_ref[...]
    logits = lax.dot_general(x, gw, (((1,), (1,)), ((), ())),
                             preferred_element_type=jnp.float32)
    iota = lax.broadcasted_iota(jnp.int32, logits.shape, 1)
    m1 = jnp.max(logits, axis=1, keepdims=True)
    i1 = jnp.min(jnp.where(logits == m1, iota, E), axis=1)
    oh1 = (iota == i1[:, None]).astype(jnp.float32)
    masked = jnp.where(oh1 > 0, -jnp.inf, logits)
    m2 = jnp.max(masked, axis=1, keepdims=True)
    i2 = jnp.min(jnp.where(masked == m2, iota, E), axis=1)
    oh2 = (iota == i2[:, None]).astype(jnp.float32)

    @pl.when(b < _NBL)
    def _pass1():
        w1 = jax.nn.sigmoid(m1 - m2)
        w1_ref[...] = jnp.broadcast_to(w1, (_RB, 16))
        w2_ref[...] = jnp.broadcast_to(1.0 - w1, (_RB, 16))

        def _bf16_bits(v):
            u = lax.bitcast_convert_type(v, jnp.int32)
            return (u + 0x7FFF + ((u >> 16) & 1)) >> 16

        blo = _bf16_bits(x[:, :D // 2]) & 0xFFFF
        bhi = _bf16_bits(x[:, D // 2:])
        xb_ref[...] = blo | (bhi << 16)
        cnt_ref[pl.ds(b, 1), :] = jnp.sum(oh1 + oh2, axis=0,
                                          keepdims=True)

    @pl.when(b >= _NBL)
    def _pass2():
        bb = b - _NBL
        rows = cnt_ref[...]
        r_iota = lax.broadcasted_iota(jnp.int32, rows.shape, 0)
        c_base = jnp.sum(jnp.where(r_iota < bb, rows, 0.0),
                         axis=0, keepdims=True)
        totals = jnp.sum(rows, axis=0, keepdims=True)
        tot_i = totals.astype(jnp.int32)
        padded = ((tot_i + BM - 1) // BM) * BM
        padded_f = padded.astype(jnp.float32)
        le_i = lax.broadcasted_iota(jnp.int32, (E, E), 0)
        le_j = lax.broadcasted_iota(jnp.int32, (E, E), 1)
        ltri8 = (le_i < le_j).astype(jnp.float32)
        pad_off = lax.dot_general(padded_f, ltri8,
                                  (((1,), (0,)), ((), ())),
                                  preferred_element_type=jnp.float32)
        tt_i = lax.broadcasted_iota(jnp.int32, (_RB, _RB), 0)
        tt_j = lax.broadcasted_iota(jnp.int32, (_RB, _RB), 1)
        strict = (tt_i > tt_j).astype(jnp.float32)
        p_strict = lax.dot_general(strict, oh1 + oh2,
                                   (((1,), (0,)), ((), ())),
                                   preferred_element_type=jnp.float32)
        m = pad_off + c_base + p_strict
        dest1 = jnp.sum(oh1 * m, axis=1).astype(jnp.int32)
        dest2 = jnp.sum(oh2 * m, axis=1).astype(jnp.int32)
        pe_ref[...] = dest1.reshape(_RB // _TPW, _TPW)
        po_ref[...] = dest2.reshape(_RB // _TPW, _TPW)

        @pl.when(b == 2 * _NBL - 1)
        def _tables():
            pad_end = pad_off + padded_f
            nb = (jnp.sum(padded_f) / BM).astype(jnp.int32)
            bi2 = lax.broadcasted_iota(jnp.int32, (NB, E), 0)
            be_raw = jnp.sum((bi2.astype(jnp.float32) * BM >=
                              pad_end).astype(jnp.int32), axis=1)
            b1 = lax.broadcasted_iota(jnp.int32, (NB,), 0)
            active = b1 < nb
            e_last = jnp.sum(jnp.where(b1 == nb - 1, be_raw, 0))
            be_ref[...] = jnp.where(active, be_raw, e_last).astype(jnp.int32)
            xbi_ref[...] = jnp.where(active, b1, nb - 1).astype(jnp.int32)
            act_ref[...] = active.astype(jnp.int32)


def _router_meta(x, gate_weight):
    wpb = _RB // _TPW
    return pl.pallas_call(
        _router_body,
        grid=(2 * _NBL,),
        in_specs=[
            pl.BlockSpec((_RB, D), lambda b: (b % _NBL, 0)),
            pl.BlockSpec((E, D), lambda b: (0, 0)),
        ],
        out_specs=[
            pl.BlockSpec((_RB, 16), lambda b: (jnp.minimum(b, _NBL - 1), 0)),
            pl.BlockSpec((_RB, 16), lambda b: (jnp.minimum(b, _NBL - 1), 0)),
            pl.BlockSpec((_RB, _DH),
                         lambda b: (jnp.minimum(b, _NBL - 1), 0)),
            pl.BlockSpec((wpb, _TPW),
                         lambda b: (jnp.maximum(b - _NBL, 0), 0)),
            pl.BlockSpec((wpb, _TPW),
                         lambda b: (jnp.maximum(b - _NBL, 0), 0)),
            pl.BlockSpec((NB,), lambda b: (0,)),
            pl.BlockSpec((NB,), lambda b: (0,)),
            pl.BlockSpec((NB,), lambda b: (0,)),
        ],
        out_shape=[
            jax.ShapeDtypeStruct((T, 16), jnp.float32),
            jax.ShapeDtypeStruct((T, 16), jnp.float32),
            jax.ShapeDtypeStruct((T, _DH), jnp.int32),
            jax.ShapeDtypeStruct((NW, _TPW), jnp.int32),
            jax.ShapeDtypeStruct((NW, _TPW), jnp.int32),
            jax.ShapeDtypeStruct((NB,), jnp.int32),
            jax.ShapeDtypeStruct((NB,), jnp.int32),
            jax.ShapeDtypeStruct((NB,), jnp.int32),
        ],
        scratch_shapes=[pltpu.VMEM((_NBL, E), jnp.float32)],
        compiler_params=pltpu.CompilerParams(
            dimension_semantics=("arbitrary",)),
    )(x, gate_weight)


def _scatter_body(xb_hbm, pe_hbm, po_hbm, xs_hbm,
                  ie_v, io_v, rows_v, s1, s2):
    wid = lax.axis_index("s") * NC + lax.axis_index("c")
    tb = wid * _TPW
    pltpu.sync_copy(pe_hbm.at[wid], ie_v)
    pltpu.sync_copy(po_hbm.at[wid], io_v)
    pltpu.sync_copy(xb_hbm.at[pl.ds(tb, _TPW)], rows_v)
    c1 = pltpu.async_copy(rows_v, xs_hbm.at[ie_v], s1)
    c2 = pltpu.async_copy(rows_v, xs_hbm.at[io_v], s2)
    c1.wait()
    c2.wait()


def _dispatch_scatter(xb, pe, po):
    mesh = plsc.VectorSubcoreMesh(core_axis_name="c", subcore_axis_name="s")
    run = pl.kernel(
        _scatter_body,
        out_type=jax.ShapeDtypeStruct((NP, _DH), jnp.int32),
        mesh=mesh,
        scratch_types=[
            pltpu.VMEM((_TPW,), jnp.int32),
            pltpu.VMEM((_TPW,), jnp.int32),
            pltpu.VMEM((_TPW, _DH), jnp.int32),
            pltpu.SemaphoreType.DMA,
            pltpu.SemaphoreType.DMA,
        ],
    )
    return run(xb, pe, po)


def _ffn_body(be_ref, xbi_ref, act_ref, xs_ref,
              wg_ref, wu_ref, wd_ref, ys_ref):
    b = pl.program_id(0)

    @pl.when(act_ref[b] == 1)
    def _():
        packed = xs_ref[...]
        xlo = lax.bitcast_convert_type(packed << 16, jnp.float32)
        xhi = lax.bitcast_convert_type(packed & jnp.int32(-65536),
                                       jnp.float32)
        xb = jnp.concatenate([xlo, xhi], axis=1).astype(jnp.bfloat16)
        wg = wg_ref[0].astype(jnp.bfloat16)
        wu = wu_ref[0].astype(jnp.bfloat16)
        wd = wd_ref[0].astype(jnp.bfloat16)
        g = jnp.dot(xb, wg, preferred_element_type=jnp.float32)
        u = jnp.dot(xb, wu, preferred_element_type=jnp.float32)
        h = (g * jax.nn.sigmoid(g)) * u
        y = jnp.dot(h.astype(jnp.bfloat16), wd,
                    preferred_element_type=jnp.float32)

        def _bf16_bits(v):
            u32 = lax.bitcast_convert_type(v, jnp.int32)
            return (u32 + 0x7FFF + ((u32 >> 16) & 1)) >> 16

        blo = _bf16_bits(y[:, :D // 2]) & 0xFFFF
        bhi = _bf16_bits(y[:, D // 2:])
        ys_ref[...] = lax.bitcast_convert_type(blo | (bhi << 16),
                                               jnp.float32)


def _grouped_ffn(xs, w_gate, w_up, w_down, be, xbi, act):
    grid_spec = pltpu.PrefetchScalarGridSpec(
        num_scalar_prefetch=3,
        grid=(NB,),
        in_specs=[
            pl.BlockSpec((BM, _DH), lambda b, be, xbi, act: (xbi[b], 0)),
            pl.BlockSpec((1, D, F), lambda b, be, xbi, act: (be[b], 0, 0)),
            pl.BlockSpec((1, D, F), lambda b, be, xbi, act: (be[b], 0, 0)),
            pl.BlockSpec((1, F, D), lambda b, be, xbi, act: (be[b], 0, 0)),
        ],
        out_specs=pl.BlockSpec((BM, _DH), lambda b, be, xbi, act: (xbi[b], 0)),
    )
    return pl.pallas_call(
        _ffn_body,
        grid_spec=grid_spec,
        out_shape=jax.ShapeDtypeStruct((NP, _DH), jnp.float32),
        compiler_params=pltpu.CompilerParams(
            dimension_semantics=("arbitrary",)),
    )(be, xbi, act, xs, w_gate, w_up, w_down)


_CCH = 16
_CNC = _TPW // _CCH
_CNB = 3


def _combine_body(ys_hbm, pe_hbm, po_hbm, w1_hbm, w2_hbm, out_hbm,
                  i1_v, i2_v, w1_v, w2_v, *scr):
    r1 = scr[:_CNB]
    r2 = scr[_CNB:2 * _CNB]
    sg1 = scr[2 * _CNB:3 * _CNB]
    sg2 = scr[3 * _CNB:4 * _CNB]
    sw1 = scr[4 * _CNB:5 * _CNB]
    sw2 = scr[5 * _CNB:]
    wid = lax.axis_index("s") * NC + lax.axis_index("c")
    base = wid * _TPW
    pltpu.sync_copy(pe_hbm.at[wid], i1_v)
    pltpu.sync_copy(po_hbm.at[wid], i2_v)
    pltpu.sync_copy(w1_hbm.at[pl.ds(base, _TPW)], w1_v)
    pltpu.sync_copy(w2_hbm.at[pl.ds(base, _TPW)], w2_v)

    def fire(c, s):
        sl = pl.ds(c * _CCH, _CCH)
        return (pltpu.async_copy(ys_hbm.at[i1_v.at[sl]], r1[s], sg1[s]),
                pltpu.async_copy(ys_hbm.at[i2_v.at[sl]], r2[s], sg2[s]))

    gd = [None] * _CNB
    wb = [None] * _CNB
    for c in range(min(_CNB - 1, _CNC)):
        gd[c] = fire(c, c)
    for c in range(_CNC):
        s = c % _CNB
        n = c + _CNB - 1
        if n < _CNC:
            sn = n % _CNB
            if wb[sn] is not None:
                wb[sn][0].wait()
                wb[sn][1].wait()
            gd[sn] = fire(n, sn)
        gd[s][0].wait()
        gd[s][1].wait()

        def add_row(r, _):
            v1 = w1_v[c * _CCH + r, pl.ds(0, 16)]
            v2 = w2_v[c * _CCH + r, pl.ds(0, 16)]
            bc = lax.bitcast_convert_type

            def add_vec(jb, _):
                for ju in range(8):
                    sl = pl.ds(jb * 128 + ju * 16, 16)
                    p1v = bc(r1[s][r, sl], jnp.int32)
                    p2v = bc(r2[s][r, sl], jnp.int32)
                    lo = (bc(p1v << 16, jnp.float32) * v1
                          + bc(p2v << 16, jnp.float32) * v2)
                    hi = (bc(p1v, jnp.float32) * v1
                          + bc(p2v, jnp.float32) * v2)
                    r1[s][r, sl] = lo
                    r2[s][r, sl] = hi
                return 0
            lax.fori_loop(0, _DH // 128, add_vec, 0)
            return 0

        lax.fori_loop(0, _CCH, add_row, 0)
        out_rows = pl.ds(base + c * _CCH, _CCH)
        wb[s] = (
            pltpu.async_copy(r1[s], out_hbm.at[out_rows, pl.ds(0, _DH)],
                             sw1[s]),
            pltpu.async_copy(r2[s], out_hbm.at[out_rows, pl.ds(_DH, _DH)],
                             sw2[s]))
    for s in range(_CNB):
        if wb[s] is not None:
            wb[s][0].wait()
            wb[s][1].wait()


def _combine(ys, pe, po, w1, w2):
    mesh = plsc.VectorSubcoreMesh(core_axis_name="c", subcore_axis_name="s")
    run = pl.kernel(
        _combine_body,
        out_type=jax.ShapeDtypeStruct((T, D), jnp.float32),
        mesh=mesh,
        scratch_types=(
            [pltpu.VMEM((_TPW,), jnp.int32)] * 2
            + [pltpu.VMEM((_TPW, 16), jnp.float32)] * 2
            + [pltpu.VMEM((_CCH, _DH), jnp.float32)] * (2 * _CNB)
            + [pltpu.SemaphoreType.DMA] * (4 * _CNB)
        ),
    )
    return run(ys, pe, po, w1, w2)


def kernel(hidden_states, gate_weight, w_gate_proj, w_up_proj, w_down_proj):
    x = hidden_states.reshape(T, D)
    w1, w2, xb, pe, po, be, xbi, act = _router_meta(x, gate_weight)
    xs = _dispatch_scatter(xb, pe, po)
    ys = _grouped_ffn(xs, w_gate_proj, w_up_proj, w_down_proj, be, xbi, act)
    out = _combine(ys, pe, po, w1, w2)
    return out.reshape(hidden_states.shape)


# speedup vs baseline: 1.1172x; 1.0425x over previous
import jax
import jax.numpy as jnp
from jax import lax
from jax.experimental import pallas as pl
from jax.experimental.pallas import tpu as pltpu
from jax.experimental.pallas import tpu_sc as plsc

T = 2048
D = 2048
E = 8
F = 768
K = 2

BM = 512
NB = 15
NP = NB * BM

NC, NS = 2, 16
NW = NC * NS
_DH = D // 2
_TPW = T // NW


_RB = 512
_NBL = T // _RB


def _router_body(x_ref, gw_ref, w1_ref, w2_ref, xb_ref, pe_ref, po_ref,
                 be_ref, xbi_ref, act_ref, cnt_ref):
    b = pl.program_id(0)
    x = x_ref[...]
    gw = gw_ref[...]
    logits = lax.dot_general(x, gw, (((1,), (1,)), ((), ())),
                             preferred_element_type=jnp.float32)
    iota = lax.broadcasted_iota(jnp.int32, logits.shape, 1)
    m1 = jnp.max(logits, axis=1, keepdims=True)
    i1 = jnp.min(jnp.where(logits == m1, iota, E), axis=1)
    oh1 = (iota == i1[:, None]).astype(jnp.float32)
    masked = jnp.where(oh1 > 0, -jnp.inf, logits)
    m2 = jnp.max(masked, axis=1, keepdims=True)
    i2 = jnp.min(jnp.where(masked == m2, iota, E), axis=1)
    oh2 = (iota == i2[:, None]).astype(jnp.float32)

    @pl.when(b < _NBL)
    def _pass1():
        w1 = jax.nn.sigmoid(m1 - m2)
        w1_ref[...] = jnp.broadcast_to(w1, (_RB, 16))
        w2_ref[...] = jnp.broadcast_to(1.0 - w1, (_RB, 16))

        def _bf16_bits(v):
            u = lax.bitcast_convert_type(v, jnp.int32)
            return (u + 0x7FFF + ((u >> 16) & 1)) >> 16

        blo = _bf16_bits(x[:, :D // 2]) & 0xFFFF
        bhi = _bf16_bits(x[:, D // 2:])
        xb_ref[...] = blo | (bhi << 16)
        cnt_ref[pl.ds(b, 1), :] = jnp.sum(oh1 + oh2, axis=0,
                                          keepdims=True)

    @pl.when(b >= _NBL)
    def _pass2():
        bb = b - _NBL
        rows = cnt_ref[...]
        r_iota = lax.broadcasted_iota(jnp.int32, rows.shape, 0)
        c_base = jnp.sum(jnp.where(r_iota < bb, rows, 0.0),
                         axis=0, keepdims=True)
        totals = jnp.sum(rows, axis=0, keepdims=True)
        tot_i = totals.astype(jnp.int32)
        padded = ((tot_i + BM - 1) // BM) * BM
        padded_f = padded.astype(jnp.float32)
        le_i = lax.broadcasted_iota(jnp.int32, (E, E), 0)
        le_j = lax.broadcasted_iota(jnp.int32, (E, E), 1)
        ltri8 = (le_i < le_j).astype(jnp.float32)
        pad_off = lax.dot_general(padded_f, ltri8,
                                  (((1,), (0,)), ((), ())),
                                  preferred_element_type=jnp.float32)
        tt_i = lax.broadcasted_iota(jnp.int32, (_RB, _RB), 0)
        tt_j = lax.broadcasted_iota(jnp.int32, (_RB, _RB), 1)
        strict = (tt_i > tt_j).astype(jnp.float32)
        p_strict = lax.dot_general(strict, oh1 + oh2,
                                   (((1,), (0,)), ((), ())),
                                   preferred_element_type=jnp.float32)
        m = pad_off + c_base + p_strict
        dest1 = jnp.sum(oh1 * m, axis=1).astype(jnp.int32)
        dest2 = jnp.sum(oh2 * m, axis=1).astype(jnp.int32)
        pe_ref[...] = dest1.reshape(_RB // _TPW, _TPW)
        po_ref[...] = dest2.reshape(_RB // _TPW, _TPW)

        @pl.when(b == 2 * _NBL - 1)
        def _tables():
            pad_end = pad_off + padded_f
            nb = (jnp.sum(padded_f) / BM).astype(jnp.int32)
            bi2 = lax.broadcasted_iota(jnp.int32, (NB, E), 0)
            be_raw = jnp.sum((bi2.astype(jnp.float32) * BM >=
                              pad_end).astype(jnp.int32), axis=1)
            b1 = lax.broadcasted_iota(jnp.int32, (NB,), 0)
            active = b1 < nb
            e_last = jnp.sum(jnp.where(b1 == nb - 1, be_raw, 0))
            be_ref[...] = jnp.where(active, be_raw, e_last).astype(jnp.int32)
            xbi_ref[...] = jnp.where(active, b1, nb - 1).astype(jnp.int32)
            act_ref[...] = active.astype(jnp.int32)


def _router_meta(x, gate_weight):
    wpb = _RB // _TPW
    return pl.pallas_call(
        _router_body,
        grid=(2 * _NBL,),
        in_specs=[
            pl.BlockSpec((_RB, D), lambda b: (b % _NBL, 0)),
            pl.BlockSpec((E, D), lambda b: (0, 0)),
        ],
        out_specs=[
            pl.BlockSpec((_RB, 16), lambda b: (jnp.minimum(b, _NBL - 1), 0)),
            pl.BlockSpec((_RB, 16), lambda b: (jnp.minimum(b, _NBL - 1), 0)),
            pl.BlockSpec((_RB, _DH),
                         lambda b: (jnp.minimum(b, _NBL - 1), 0)),
            pl.BlockSpec((wpb, _TPW),
                         lambda b: (jnp.maximum(b - _NBL, 0), 0)),
            pl.BlockSpec((wpb, _TPW),
                         lambda b: (jnp.maximum(b - _NBL, 0), 0)),
            pl.BlockSpec((NB,), lambda b: (0,)),
            pl.BlockSpec((NB,), lambda b: (0,)),
            pl.BlockSpec((NB,), lambda b: (0,)),
        ],
        out_shape=[
            jax.ShapeDtypeStruct((T, 16), jnp.float32),
            jax.ShapeDtypeStruct((T, 16), jnp.float32),
            jax.ShapeDtypeStruct((T, _DH), jnp.int32),
            jax.ShapeDtypeStruct((NW, _TPW), jnp.int32),
            jax.ShapeDtypeStruct((NW, _TPW), jnp.int32),
            jax.ShapeDtypeStruct((NB,), jnp.int32),
            jax.ShapeDtypeStruct((NB,), jnp.int32),
            jax.ShapeDtypeStruct((NB,), jnp.int32),
        ],
        scratch_shapes=[pltpu.VMEM((_NBL, E), jnp.float32)],
        compiler_params=pltpu.CompilerParams(
            dimension_semantics=("arbitrary",)),
    )(x, gate_weight)


def _scatter_body(xb_hbm, pe_hbm, po_hbm, xs_hbm,
                  ie_v, io_v, rows_v, s1, s2):
    wid = lax.axis_index("s") * NC + lax.axis_index("c")
    tb = wid * _TPW
    pltpu.sync_copy(pe_hbm.at[wid], ie_v)
    pltpu.sync_copy(po_hbm.at[wid], io_v)
    pltpu.sync_copy(xb_hbm.at[pl.ds(tb, _TPW)], rows_v)
    c1 = pltpu.async_copy(rows_v, xs_hbm.at[ie_v], s1)
    c2 = pltpu.async_copy(rows_v, xs_hbm.at[io_v], s2)
    c1.wait()
    c2.wait()


def _dispatch_scatter(xb, pe, po):
    mesh = plsc.VectorSubcoreMesh(core_axis_name="c", subcore_axis_name="s")
    run = pl.kernel(
        _scatter_body,
        out_type=jax.ShapeDtypeStruct((NP, _DH), jnp.int32),
        mesh=mesh,
        scratch_types=[
            pltpu.VMEM((_TPW,), jnp.int32),
            pltpu.VMEM((_TPW,), jnp.int32),
            pltpu.VMEM((_TPW, _DH), jnp.int32),
            pltpu.SemaphoreType.DMA,
            pltpu.SemaphoreType.DMA,
        ],
    )
    return run(xb, pe, po)


def _ffn_body(be_ref, xbi_ref, act_ref, xs_ref,
              wg_ref, wu_ref, wd_ref, ys_ref):
    b = pl.program_id(0)

    @pl.when(act_ref[b] == 1)
    def _():
        packed = xs_ref[...]
        xlo = lax.bitcast_convert_type(packed << 16, jnp.float32)
        xhi = lax.bitcast_convert_type(packed & jnp.int32(-65536),
                                       jnp.float32)
        xb = jnp.concatenate([xlo, xhi], axis=1).astype(jnp.bfloat16)
        wg = wg_ref[0].astype(jnp.bfloat16)
        wu = wu_ref[0].astype(jnp.bfloat16)
        wd = wd_ref[0].astype(jnp.bfloat16)
        g = jnp.dot(xb, wg, preferred_element_type=jnp.float32)
        u = jnp.dot(xb, wu, preferred_element_type=jnp.float32)
        h = (g * jax.nn.sigmoid(g)) * u
        y = jnp.dot(h.astype(jnp.bfloat16), wd,
                    preferred_element_type=jnp.float32)

        def _bf16_bits(v):
            u32 = lax.bitcast_convert_type(v, jnp.int32)
            return (u32 + 0x7FFF + ((u32 >> 16) & 1)) >> 16

        blo = _bf16_bits(y[:, :D // 2]) & 0xFFFF
        bhi = _bf16_bits(y[:, D // 2:])
        ys_ref[...] = lax.bitcast_convert_type(blo | (bhi << 16),
                                               jnp.float32)


def _grouped_ffn(xs, w_gate, w_up, w_down, be, xbi, act):
    grid_spec = pltpu.PrefetchScalarGridSpec(
        num_scalar_prefetch=3,
        grid=(NB,),
        in_specs=[
            pl.BlockSpec((BM, _DH), lambda b, be, xbi, act: (xbi[b], 0)),
            pl.BlockSpec((1, D, F), lambda b, be, xbi, act: (be[b], 0, 0)),
            pl.BlockSpec((1, D, F), lambda b, be, xbi, act: (be[b], 0, 0)),
            pl.BlockSpec((1, F, D), lambda b, be, xbi, act: (be[b], 0, 0)),
        ],
        out_specs=pl.BlockSpec((BM, _DH), lambda b, be, xbi, act: (xbi[b], 0)),
    )
    return pl.pallas_call(
        _ffn_body,
        grid_spec=grid_spec,
        out_shape=jax.ShapeDtypeStruct((NP, _DH), jnp.float32),
        compiler_params=pltpu.CompilerParams(
            dimension_semantics=("arbitrary",)),
    )(be, xbi, act, xs, w_gate, w_up, w_down)


_CCH = 16
_CNC = _TPW // _CCH
_CNB = 3


def _combine_body(ys_hbm, pe_hbm, po_hbm, w1_hbm, w2_hbm, out_hbm,
                  i1_v, i2_v, w1_v, w2_v, *scr):
    r1 = scr[:_CNB]
    r2 = scr[_CNB:2 * _CNB]
    sg1 = scr[2 * _CNB:3 * _CNB]
    sg2 = scr[3 * _CNB:4 * _CNB]
    sw1 = scr[4 * _CNB:5 * _CNB]
    sw2 = scr[5 * _CNB:]
    wid = lax.axis_index("s") * NC + lax.axis_index("c")
    base = wid * _TPW
    pltpu.sync_copy(pe_hbm.at[wid], i1_v)
    pltpu.sync_copy(po_hbm.at[wid], i2_v)
    pltpu.sync_copy(w1_hbm.at[pl.ds(base, _TPW)], w1_v)
    pltpu.sync_copy(w2_hbm.at[pl.ds(base, _TPW)], w2_v)

    def fire(c, s):
        sl = pl.ds(c * _CCH, _CCH)
        return (pltpu.async_copy(ys_hbm.at[i1_v.at[sl]], r1[s], sg1[s]),
                pltpu.async_copy(ys_hbm.at[i2_v.at[sl]], r2[s], sg2[s]))

    gd = [None] * _CNB
    wb = [None] * _CNB
    for c in range(min(_CNB - 1, _CNC)):
        gd[c] = fire(c, c)
    for c in range(_CNC):
        s = c % _CNB
        n = c + _CNB - 1
        if n < _CNC:
            sn = n % _CNB
            if wb[sn] is not None:
                wb[sn][0].wait()
                wb[sn][1].wait()
            gd[sn] = fire(n, sn)
        gd[s][0].wait()
        gd[s][1].wait()

        def add_row(r, _):
            v1 = w1_v[c * _CCH + r, pl.ds(0, 16)]
            v2 = w2_v[c * _CCH + r, pl.ds(0, 16)]
            bc = lax.bitcast_convert_type

            def add_vec(jb, _):
                for ju in range(8):
                    sl = pl.ds(jb * 128 + ju * 16, 16)
                    p1v = bc(r1[s][r, sl], jnp.int32)
                    p2v = bc(r2[s][r, sl], jnp.int32)
                    lo = (bc(p1v << 16, jnp.float32) * v1
                          + bc(p2v << 16, jnp.float32) * v2)
                    hi = (bc(p1v, jnp.float32) * v1
                          + bc(p2v, jnp.float32) * v2)
                    r1[s][r, sl] = lo
                    r2[s][r, sl] = hi
                return 0
            lax.fori_loop(0, _DH // 128, add_vec, 0)
            return 0

        lax.fori_loop(0, _CCH, add_row, 0)
        out_rows = pl.ds(base + c * _CCH, _CCH)
        wb[s] = (
            pltpu.async_copy(r1[s], out_hbm.at[out_rows, pl.ds(0, _DH)],
                             sw1[s]),
            pltpu.async_copy(r2[s], out_hbm.at[out_rows, pl.ds(_DH, _DH)],
                             sw2[s]))
    for s in range(_CNB):
        if wb[s] is not None:
            wb[s][0].wait()
            wb[s][1].wait()


def _combine(ys, pe, po, w1, w2):
    mesh = plsc.VectorSubcoreMesh(core_axis_name="c", subcore_axis_name="s")
    run = pl.kernel(
        _combine_body,
        out_type=jax.ShapeDtypeStruct((T, D), jnp.float32),
        mesh=mesh,
        scratch_types=(
            [pltpu.VMEM((_TPW,), jnp.int32)] * 2
            + [pltpu.VMEM((_TPW, 16), jnp.float32)] * 2
            + [pltpu.VMEM((_CCH, _DH), jnp.float32)] * (2 * _CNB)
            + [pltpu.SemaphoreType.DMA] * (4 * _CNB)
        ),
    )
    return run(ys, pe, po, w1, w2)


def kernel(hidden_states, gate_weight, w_gate_proj, w_up_proj, w_down_proj):
    x = hidden_states.reshape(T, D)
    w1, w2, xb, pe, po, be, xbi, act = _router_meta(x, gate_weight)
    xs = _dispatch_scatter(xb, pe, po)
    ys = _grouped_ffn(xs, w_gate_proj, w_up_proj, w_down_proj, be, xbi, act)
    out = _combine(ys, pe, po, w1, w2)
    return out.reshape(hidden_states.shape)


# speedup vs baseline: 1.1492x; 1.0287x over previous
import jax
import jax.numpy as jnp
from jax import lax
from jax.experimental import pallas as pl
from jax.experimental.pallas import tpu as pltpu
from jax.experimental.pallas import tpu_sc as plsc

T = 2048
D = 2048
E = 8
F = 768
K = 2

BM = 512
NB = 15
NP = NB * BM

NC, NS = 2, 16
NW = NC * NS
_DH = D // 2
_TPW = T // NW


_RB = 512
_NBL = T // _RB


def _router_body(x_ref, gw_ref, w1_ref, w2_ref, xb_ref, pe_ref, po_ref,
                 be_ref, xbi_ref, act_ref, cnt_ref, lg_ref):
    b = pl.program_id(0)

    def _top2(logits):
        iota = lax.broadcasted_iota(jnp.int32, logits.shape, 1)
        m1 = jnp.max(logits, axis=1, keepdims=True)
        i1 = jnp.min(jnp.where(logits == m1, iota, E), axis=1)
        oh1 = (iota == i1[:, None]).astype(jnp.float32)
        masked = jnp.where(oh1 > 0, -jnp.inf, logits)
        m2 = jnp.max(masked, axis=1, keepdims=True)
        i2 = jnp.min(jnp.where(masked == m2, iota, E), axis=1)
        oh2 = (iota == i2[:, None]).astype(jnp.float32)
        return m1, m2, oh1, oh2

    @pl.when(b < _NBL)
    def _pass1():
        x = x_ref[...]
        gw = gw_ref[...]
        logits = lax.dot_general(x, gw, (((1,), (1,)), ((), ())),
                                 preferred_element_type=jnp.float32)
        lg_ref[pl.ds(b * _RB, _RB), :] = logits
        m1, m2, oh1, oh2 = _top2(logits)
        w1 = jax.nn.sigmoid(m1 - m2)
        w1_ref[...] = jnp.broadcast_to(w1, (_RB, 16))
        w2_ref[...] = jnp.broadcast_to(1.0 - w1, (_RB, 16))

        def _bf16_bits(v):
            u = lax.bitcast_convert_type(v, jnp.int32)
            return (u + 0x7FFF + ((u >> 16) & 1)) >> 16

        blo = _bf16_bits(x[:, :D // 2]) & 0xFFFF
        bhi = _bf16_bits(x[:, D // 2:])
        xb_ref[...] = blo | (bhi << 16)
        cnt_ref[pl.ds(b, 1), :] = jnp.sum(oh1 + oh2, axis=0,
                                          keepdims=True)

    @pl.when(b >= _NBL)
    def _pass2():
        bb = b - _NBL
        _, _, oh1, oh2 = _top2(lg_ref[pl.ds(bb * _RB, _RB), :])
        rows = cnt_ref[...]
        r_iota = lax.broadcasted_iota(jnp.int32, rows.shape, 0)
        c_base = jnp.sum(jnp.where(r_iota < bb, rows, 0.0),
                         axis=0, keepdims=True)
        totals = jnp.sum(rows, axis=0, keepdims=True)
        tot_i = totals.astype(jnp.int32)
        padded = ((tot_i + BM - 1) // BM) * BM
        padded_f = padded.astype(jnp.float32)
        le_i = lax.broadcasted_iota(jnp.int32, (E, E), 0)
        le_j = lax.broadcasted_iota(jnp.int32, (E, E), 1)
        ltri8 = (le_i < le_j).astype(jnp.float32)
        pad_off = lax.dot_general(padded_f, ltri8,
                                  (((1,), (0,)), ((), ())),
                                  preferred_element_type=jnp.float32)
        tt_i = lax.broadcasted_iota(jnp.int32, (_RB, _RB), 0)
        tt_j = lax.broadcasted_iota(jnp.int32, (_RB, _RB), 1)
        strict = (tt_i > tt_j).astype(jnp.float32)
        p_strict = lax.dot_general(strict, oh1 + oh2,
                                   (((1,), (0,)), ((), ())),
                                   preferred_element_type=jnp.float32)
        m = pad_off + c_base + p_strict
        dest1 = jnp.sum(oh1 * m, axis=1).astype(jnp.int32)
        dest2 = jnp.sum(oh2 * m, axis=1).astype(jnp.int32)
        pe_ref[...] = dest1.reshape(_RB // _TPW, _TPW)
        po_ref[...] = dest2.reshape(_RB // _TPW, _TPW)

        @pl.when(b == 2 * _NBL - 1)
        def _tables():
            pad_end = pad_off + padded_f
            nb = (jnp.sum(padded_f) / BM).astype(jnp.int32)
            bi2 = lax.broadcasted_iota(jnp.int32, (NB, E), 0)
            be_raw = jnp.sum((bi2.astype(jnp.float32) * BM >=
                              pad_end).astype(jnp.int32), axis=1)
            b1 = lax.broadcasted_iota(jnp.int32, (NB,), 0)
            active = b1 < nb
            e_last = jnp.sum(jnp.where(b1 == nb - 1, be_raw, 0))
            be_ref[...] = jnp.where(active, be_raw, e_last).astype(jnp.int32)
            xbi_ref[...] = jnp.where(active, b1, nb - 1).astype(jnp.int32)
            act_ref[...] = active.astype(jnp.int32)


def _router_meta(x, gate_weight):
    wpb = _RB // _TPW
    return pl.pallas_call(
        _router_body,
        grid=(2 * _NBL,),
        in_specs=[
            pl.BlockSpec((_RB, D), lambda b: (jnp.minimum(b, _NBL - 1), 0)),
            pl.BlockSpec((E, D), lambda b: (0, 0)),
        ],
        out_specs=[
            pl.BlockSpec((_RB, 16), lambda b: (jnp.minimum(b, _NBL - 1), 0)),
            pl.BlockSpec((_RB, 16), lambda b: (jnp.minimum(b, _NBL - 1), 0)),
            pl.BlockSpec((_RB, _DH),
                         lambda b: (jnp.minimum(b, _NBL - 1), 0)),
            pl.BlockSpec((wpb, _TPW),
                         lambda b: (jnp.maximum(b - _NBL, 0), 0)),
            pl.BlockSpec((wpb, _TPW),
                         lambda b: (jnp.maximum(b - _NBL, 0), 0)),
            pl.BlockSpec((NB,), lambda b: (0,)),
            pl.BlockSpec((NB,), lambda b: (0,)),
            pl.BlockSpec((NB,), lambda b: (0,)),
        ],
        out_shape=[
            jax.ShapeDtypeStruct((T, 16), jnp.float32),
            jax.ShapeDtypeStruct((T, 16), jnp.float32),
            jax.ShapeDtypeStruct((T, _DH), jnp.int32),
            jax.ShapeDtypeStruct((NW, _TPW), jnp.int32),
            jax.ShapeDtypeStruct((NW, _TPW), jnp.int32),
            jax.ShapeDtypeStruct((NB,), jnp.int32),
            jax.ShapeDtypeStruct((NB,), jnp.int32),
            jax.ShapeDtypeStruct((NB,), jnp.int32),
        ],
        scratch_shapes=[pltpu.VMEM((_NBL, E), jnp.float32),
                        pltpu.VMEM((T, E), jnp.float32)],
        compiler_params=pltpu.CompilerParams(
            dimension_semantics=("arbitrary",)),
    )(x, gate_weight)


def _scatter_body(xb_hbm, pe_hbm, po_hbm, xs_hbm,
                  ie_v, io_v, rows_v, s1, s2):
    wid = lax.axis_index("s") * NC + lax.axis_index("c")
    tb = wid * _TPW
    pltpu.sync_copy(pe_hbm.at[wid], ie_v)
    pltpu.sync_copy(po_hbm.at[wid], io_v)
    pltpu.sync_copy(xb_hbm.at[pl.ds(tb, _TPW)], rows_v)
    c1 = pltpu.async_copy(rows_v, xs_hbm.at[ie_v], s1)
    c2 = pltpu.async_copy(rows_v, xs_hbm.at[io_v], s2)
    c1.wait()
    c2.wait()


def _dispatch_scatter(xb, pe, po):
    mesh = plsc.VectorSubcoreMesh(core_axis_name="c", subcore_axis_name="s")
    run = pl.kernel(
        _scatter_body,
        out_type=jax.ShapeDtypeStruct((NP, _DH), jnp.int32),
        mesh=mesh,
        scratch_types=[
            pltpu.VMEM((_TPW,), jnp.int32),
            pltpu.VMEM((_TPW,), jnp.int32),
            pltpu.VMEM((_TPW, _DH), jnp.int32),
            pltpu.SemaphoreType.DMA,
            pltpu.SemaphoreType.DMA,
        ],
    )
    return run(xb, pe, po)


def _ffn_body(be_ref, xbi_ref, act_ref, xs_ref,
              wg_ref, wu_ref, wd_ref, ys_ref):
    b = pl.program_id(0)

    @pl.when(act_ref[b] == 1)
    def _():
        packed = xs_ref[...]
        xlo = lax.bitcast_convert_type(packed << 16, jnp.float32)
        xhi = lax.bitcast_convert_type(packed & jnp.int32(-65536),
                                       jnp.float32)
        xb = jnp.concatenate([xlo, xhi], axis=1).astype(jnp.bfloat16)
        wg = wg_ref[0].astype(jnp.bfloat16)
        wu = wu_ref[0].astype(jnp.bfloat16)
        wd = wd_ref[0].astype(jnp.bfloat16)
        g = jnp.dot(xb, wg, preferred_element_type=jnp.float32)
        u = jnp.dot(xb, wu, preferred_element_type=jnp.float32)
        h = (g * jax.nn.sigmoid(g)) * u
        y = jnp.dot(h.astype(jnp.bfloat16), wd,
                    preferred_element_type=jnp.float32)

        def _bf16_bits(v):
            u32 = lax.bitcast_convert_type(v, jnp.int32)
            return (u32 + 0x7FFF + ((u32 >> 16) & 1)) >> 16

        blo = _bf16_bits(y[:, :D // 2]) & 0xFFFF
        bhi = _bf16_bits(y[:, D // 2:])
        ys_ref[...] = lax.bitcast_convert_type(blo | (bhi << 16),
                                               jnp.float32)


def _grouped_ffn(xs, w_gate, w_up, w_down, be, xbi, act):
    grid_spec = pltpu.PrefetchScalarGridSpec(
        num_scalar_prefetch=3,
        grid=(NB,),
        in_specs=[
            pl.BlockSpec((BM, _DH), lambda b, be, xbi, act: (xbi[b], 0)),
            pl.BlockSpec((1, D, F), lambda b, be, xbi, act: (be[b], 0, 0)),
            pl.BlockSpec((1, D, F), lambda b, be, xbi, act: (be[b], 0, 0)),
            pl.BlockSpec((1, F, D), lambda b, be, xbi, act: (be[b], 0, 0)),
        ],
        out_specs=pl.BlockSpec((BM, _DH), lambda b, be, xbi, act: (xbi[b], 0)),
    )
    return pl.pallas_call(
        _ffn_body,
        grid_spec=grid_spec,
        out_shape=jax.ShapeDtypeStruct((NP, _DH), jnp.float32),
        compiler_params=pltpu.CompilerParams(
            dimension_semantics=("arbitrary",)),
    )(be, xbi, act, xs, w_gate, w_up, w_down)


_CCH = 16
_CNC = _TPW // _CCH
_CNB = 3


def _combine_body(ys_hbm, pe_hbm, po_hbm, w1_hbm, w2_hbm, out_hbm,
                  i1_v, i2_v, w1_v, w2_v, *scr):
    r1 = scr[:_CNB]
    r2 = scr[_CNB:2 * _CNB]
    sg1 = scr[2 * _CNB:3 * _CNB]
    sg2 = scr[3 * _CNB:4 * _CNB]
    sw1 = scr[4 * _CNB:5 * _CNB]
    sw2 = scr[5 * _CNB:]
    wid = lax.axis_index("s") * NC + lax.axis_index("c")
    base = wid * _TPW
    pltpu.sync_copy(pe_hbm.at[wid], i1_v)
    pltpu.sync_copy(po_hbm.at[wid], i2_v)
    pltpu.sync_copy(w1_hbm.at[pl.ds(base, _TPW)], w1_v)
    pltpu.sync_copy(w2_hbm.at[pl.ds(base, _TPW)], w2_v)

    def fire(c, s):
        sl = pl.ds(c * _CCH, _CCH)
        return (pltpu.async_copy(ys_hbm.at[i1_v.at[sl]], r1[s], sg1[s]),
                pltpu.async_copy(ys_hbm.at[i2_v.at[sl]], r2[s], sg2[s]))

    gd = [None] * _CNB
    wb = [None] * _CNB
    for c in range(min(_CNB - 1, _CNC)):
        gd[c] = fire(c, c)
    for c in range(_CNC):
        s = c % _CNB
        n = c + _CNB - 1
        if n < _CNC:
            sn = n % _CNB
            if wb[sn] is not None:
                wb[sn][0].wait()
                wb[sn][1].wait()
            gd[sn] = fire(n, sn)
        gd[s][0].wait()
        gd[s][1].wait()

        def add_row(r, _):
            v1 = w1_v[c * _CCH + r, pl.ds(0, 16)]
            v2 = w2_v[c * _CCH + r, pl.ds(0, 16)]
            bc = lax.bitcast_convert_type

            def add_vec(jb, _):
                for ju in range(8):
                    sl = pl.ds(jb * 128 + ju * 16, 16)
                    p1v = bc(r1[s][r, sl], jnp.int32)
                    p2v = bc(r2[s][r, sl], jnp.int32)
                    lo = (bc(p1v << 16, jnp.float32) * v1
                          + bc(p2v << 16, jnp.float32) * v2)
                    hi = (bc(p1v, jnp.float32) * v1
                          + bc(p2v, jnp.float32) * v2)
                    r1[s][r, sl] = lo
                    r2[s][r, sl] = hi
                return 0
            lax.fori_loop(0, _DH // 128, add_vec, 0)
            return 0

        lax.fori_loop(0, _CCH, add_row, 0)
        out_rows = pl.ds(base + c * _CCH, _CCH)
        wb[s] = (
            pltpu.async_copy(r1[s], out_hbm.at[out_rows, pl.ds(0, _DH)],
                             sw1[s]),
            pltpu.async_copy(r2[s], out_hbm.at[out_rows, pl.ds(_DH, _DH)],
                             sw2[s]))
    for s in range(_CNB):
        if wb[s] is not None:
            wb[s][0].wait()
            wb[s][1].wait()


def _combine(ys, pe, po, w1, w2):
    mesh = plsc.VectorSubcoreMesh(core_axis_name="c", subcore_axis_name="s")
    run = pl.kernel(
        _combine_body,
        out_type=jax.ShapeDtypeStruct((T, D), jnp.float32),
        mesh=mesh,
        scratch_types=(
            [pltpu.VMEM((_TPW,), jnp.int32)] * 2
            + [pltpu.VMEM((_TPW, 16), jnp.float32)] * 2
            + [pltpu.VMEM((_CCH, _DH), jnp.float32)] * (2 * _CNB)
            + [pltpu.SemaphoreType.DMA] * (4 * _CNB)
        ),
    )
    return run(ys, pe, po, w1, w2)


def kernel(hidden_states, gate_weight, w_gate_proj, w_up_proj, w_down_proj):
    x = hidden_states.reshape(T, D)
    w1, w2, xb, pe, po, be, xbi, act = _router_meta(x, gate_weight)
    xs = _dispatch_scatter(xb, pe, po)
    ys = _grouped_ffn(xs, w_gate_proj, w_up_proj, w_down_proj, be, xbi, act)
    out = _combine(ys, pe, po, w1, w2)
    return out.reshape(hidden_states.shape)


# speedup vs baseline: 1.1594x; 1.0089x over previous
import jax
import jax.numpy as jnp
from jax import lax
from jax.experimental import pallas as pl
from jax.experimental.pallas import tpu as pltpu
from jax.experimental.pallas import tpu_sc as plsc

T = 2048
D = 2048
E = 8
F = 768
K = 2

BM = 512
NB = 15
NP = NB * BM

NC, NS = 2, 16
NW = NC * NS
_DH = D // 2
_TPW = T // NW


_RB = 512
_NBL = T // _RB


def _router_body(x_ref, gw_ref, w1_ref, w2_ref, xb_ref, pe_ref, po_ref,
                 be_ref, xbi_ref, act_ref, cnt_ref, lg_ref):
    b = pl.program_id(0)

    def _top2(logits):
        iota = lax.broadcasted_iota(jnp.int32, logits.shape, 1)
        m1 = jnp.max(logits, axis=1, keepdims=True)
        i1 = jnp.min(jnp.where(logits == m1, iota, E), axis=1)
        oh1 = (iota == i1[:, None]).astype(jnp.float32)
        masked = jnp.where(oh1 > 0, -jnp.inf, logits)
        m2 = jnp.max(masked, axis=1, keepdims=True)
        i2 = jnp.min(jnp.where(masked == m2, iota, E), axis=1)
        oh2 = (iota == i2[:, None]).astype(jnp.float32)
        return m1, m2, oh1, oh2

    @pl.when(b < _NBL)
    def _pass1():
        x = x_ref[...]
        gw = gw_ref[...]
        logits = lax.dot_general(x, gw, (((1,), (1,)), ((), ())),
                                 preferred_element_type=jnp.float32)
        lg_ref[pl.ds(b * _RB, _RB), :] = logits
        m1, m2, oh1, oh2 = _top2(logits)
        w1 = jax.nn.sigmoid(m1 - m2)
        w1_ref[...] = jnp.broadcast_to(w1, (_RB, 16))
        w2_ref[...] = jnp.broadcast_to(1.0 - w1, (_RB, 16))

        def _bf16_bits(v):
            u = lax.bitcast_convert_type(v, jnp.int32)
            return (u + 0x7FFF + ((u >> 16) & 1)) >> 16

        blo = _bf16_bits(x[:, :D // 2]) & 0xFFFF
        bhi = _bf16_bits(x[:, D // 2:])
        xb_ref[...] = blo | (bhi << 16)
        cnt_ref[pl.ds(b, 1), :] = jnp.sum(oh1 + oh2, axis=0,
                                          keepdims=True)

    @pl.when(b >= _NBL)
    def _pass2():
        bb = b - _NBL
        _, _, oh1, oh2 = _top2(lg_ref[pl.ds(bb * _RB, _RB), :])
        rows = cnt_ref[...]
        r_iota = lax.broadcasted_iota(jnp.int32, rows.shape, 0)
        c_base = jnp.sum(jnp.where(r_iota < bb, rows, 0.0),
                         axis=0, keepdims=True)
        totals = jnp.sum(rows, axis=0, keepdims=True)
        tot_i = totals.astype(jnp.int32)
        padded = ((tot_i + BM - 1) // BM) * BM
        padded_f = padded.astype(jnp.float32)
        le_i = lax.broadcasted_iota(jnp.int32, (E, E), 0)
        le_j = lax.broadcasted_iota(jnp.int32, (E, E), 1)
        ltri8 = (le_i < le_j).astype(jnp.float32)
        pad_off = lax.dot_general(padded_f, ltri8,
                                  (((1,), (0,)), ((), ())),
                                  preferred_element_type=jnp.float32)
        tt_i = lax.broadcasted_iota(jnp.int32, (_RB, _RB), 0)
        tt_j = lax.broadcasted_iota(jnp.int32, (_RB, _RB), 1)
        strict = (tt_i > tt_j).astype(jnp.float32)
        p_strict = lax.dot_general(strict, oh1 + oh2,
                                   (((1,), (0,)), ((), ())),
                                   preferred_element_type=jnp.float32)
        m = pad_off + c_base + p_strict
        dest1 = jnp.sum(oh1 * m, axis=1).astype(jnp.int32)
        dest2 = jnp.sum(oh2 * m, axis=1).astype(jnp.int32)
        pe_ref[...] = dest1.reshape(_RB // _TPW, _TPW)
        po_ref[...] = dest2.reshape(_RB // _TPW, _TPW)

        @pl.when(b == 2 * _NBL - 1)
        def _tables():
            pad_end = pad_off + padded_f
            nb = (jnp.sum(padded_f) / BM).astype(jnp.int32)
            bi2 = lax.broadcasted_iota(jnp.int32, (NB, E), 0)
            be_raw = jnp.sum((bi2.astype(jnp.float32) * BM >=
                              pad_end).astype(jnp.int32), axis=1)
            b1 = lax.broadcasted_iota(jnp.int32, (NB,), 0)
            active = b1 < nb
            e_last = jnp.sum(jnp.where(b1 == nb - 1, be_raw, 0))
            be_ref[...] = jnp.where(active, be_raw, e_last).astype(jnp.int32)
            xbi_ref[...] = jnp.where(active, b1, nb - 1).astype(jnp.int32)
            act_ref[...] = active.astype(jnp.int32)


def _router_meta(x, gate_weight):
    wpb = _RB // _TPW
    return pl.pallas_call(
        _router_body,
        grid=(2 * _NBL,),
        in_specs=[
            pl.BlockSpec((_RB, D), lambda b: (jnp.minimum(b, _NBL - 1), 0)),
            pl.BlockSpec((E, D), lambda b: (0, 0)),
        ],
        out_specs=[
            pl.BlockSpec((_RB, 16), lambda b: (jnp.minimum(b, _NBL - 1), 0)),
            pl.BlockSpec((_RB, 16), lambda b: (jnp.minimum(b, _NBL - 1), 0)),
            pl.BlockSpec((_RB, _DH),
                         lambda b: (jnp.minimum(b, _NBL - 1), 0)),
            pl.BlockSpec((wpb, _TPW),
                         lambda b: (jnp.maximum(b - _NBL, 0), 0)),
            pl.BlockSpec((wpb, _TPW),
                         lambda b: (jnp.maximum(b - _NBL, 0), 0)),
            pl.BlockSpec((NB,), lambda b: (0,)),
            pl.BlockSpec((NB,), lambda b: (0,)),
            pl.BlockSpec((NB,), lambda b: (0,)),
        ],
        out_shape=[
            jax.ShapeDtypeStruct((T, 16), jnp.float32),
            jax.ShapeDtypeStruct((T, 16), jnp.float32),
            jax.ShapeDtypeStruct((T, _DH), jnp.int32),
            jax.ShapeDtypeStruct((NW, _TPW), jnp.int32),
            jax.ShapeDtypeStruct((NW, _TPW), jnp.int32),
            jax.ShapeDtypeStruct((NB,), jnp.int32),
            jax.ShapeDtypeStruct((NB,), jnp.int32),
            jax.ShapeDtypeStruct((NB,), jnp.int32),
        ],
        scratch_shapes=[pltpu.VMEM((_NBL, E), jnp.float32),
                        pltpu.VMEM((T, E), jnp.float32)],
        compiler_params=pltpu.CompilerParams(
            dimension_semantics=("arbitrary",)),
    )(x, gate_weight)


def _scatter_body(xb_hbm, pe_hbm, po_hbm, xs_hbm,
                  ie_v, io_v, rows_v, s1, s2):
    wid = lax.axis_index("s") * NC + lax.axis_index("c")
    tb = wid * _TPW
    pltpu.sync_copy(pe_hbm.at[wid], ie_v)
    pltpu.sync_copy(po_hbm.at[wid], io_v)
    pltpu.sync_copy(xb_hbm.at[pl.ds(tb, _TPW)], rows_v)
    c1 = pltpu.async_copy(rows_v, xs_hbm.at[ie_v], s1)
    c2 = pltpu.async_copy(rows_v, xs_hbm.at[io_v], s2)
    c1.wait()
    c2.wait()


def _dispatch_scatter(xb, pe, po):
    mesh = plsc.VectorSubcoreMesh(core_axis_name="c", subcore_axis_name="s")
    run = pl.kernel(
        _scatter_body,
        out_type=jax.ShapeDtypeStruct((NP, _DH), jnp.int32),
        mesh=mesh,
        scratch_types=[
            pltpu.VMEM((_TPW,), jnp.int32),
            pltpu.VMEM((_TPW,), jnp.int32),
            pltpu.VMEM((_TPW, _DH), jnp.int32),
            pltpu.SemaphoreType.DMA,
            pltpu.SemaphoreType.DMA,
        ],
    )
    return run(xb, pe, po)


def _ffn_body(be_ref, xbi_ref, act_ref, xs_ref,
              wg_ref, wu_ref, wd_ref, ys_ref):
    b = pl.program_id(0)

    @pl.when(act_ref[b] == 1)
    def _():
        packed = xs_ref[...]
        xlo = lax.bitcast_convert_type(packed << 16, jnp.float32)
        xhi = lax.bitcast_convert_type(packed & jnp.int32(-65536),
                                       jnp.float32)
        xb = jnp.concatenate([xlo, xhi], axis=1).astype(jnp.bfloat16)
        wg = wg_ref[0].astype(jnp.bfloat16)
        wu = wu_ref[0].astype(jnp.bfloat16)
        wd = wd_ref[0].astype(jnp.bfloat16)
        g = jnp.dot(xb, wg, preferred_element_type=jnp.float32)
        u = jnp.dot(xb, wu, preferred_element_type=jnp.float32)
        h = (g * jax.nn.sigmoid(g)) * u
        y = jnp.dot(h.astype(jnp.bfloat16), wd,
                    preferred_element_type=jnp.float32)

        def _bf16_bits(v):
            u32 = lax.bitcast_convert_type(v, jnp.int32)
            return (u32 + 0x7FFF + ((u32 >> 16) & 1)) >> 16

        blo = _bf16_bits(y[:, :D // 2]) & 0xFFFF
        bhi = _bf16_bits(y[:, D // 2:])
        ys_ref[...] = lax.bitcast_convert_type(blo | (bhi << 16),
                                               jnp.float32)


def _grouped_ffn(xs, w_gate, w_up, w_down, be, xbi, act):
    grid_spec = pltpu.PrefetchScalarGridSpec(
        num_scalar_prefetch=3,
        grid=(NB,),
        in_specs=[
            pl.BlockSpec((BM, _DH), lambda b, be, xbi, act: (xbi[b], 0)),
            pl.BlockSpec((1, D, F), lambda b, be, xbi, act: (be[b], 0, 0)),
            pl.BlockSpec((1, D, F), lambda b, be, xbi, act: (be[b], 0, 0)),
            pl.BlockSpec((1, F, D), lambda b, be, xbi, act: (be[b], 0, 0)),
        ],
        out_specs=pl.BlockSpec((BM, _DH), lambda b, be, xbi, act: (xbi[b], 0)),
    )
    return pl.pallas_call(
        _ffn_body,
        grid_spec=grid_spec,
        out_shape=jax.ShapeDtypeStruct((NP, _DH), jnp.float32),
        compiler_params=pltpu.CompilerParams(
            dimension_semantics=("arbitrary",)),
    )(be, xbi, act, xs, w_gate, w_up, w_down)


_CCH = 8
_CNC = _TPW // _CCH
_CNB = 6


def _combine_body(ys_hbm, pe_hbm, po_hbm, w1_hbm, w2_hbm, out_hbm,
                  i1_v, i2_v, w1_v, w2_v, *scr):
    r1 = scr[:_CNB]
    r2 = scr[_CNB:2 * _CNB]
    sg1 = scr[2 * _CNB:3 * _CNB]
    sg2 = scr[3 * _CNB:4 * _CNB]
    sw1 = scr[4 * _CNB:5 * _CNB]
    sw2 = scr[5 * _CNB:]
    wid = lax.axis_index("s") * NC + lax.axis_index("c")
    base = wid * _TPW
    pltpu.sync_copy(pe_hbm.at[wid], i1_v)
    pltpu.sync_copy(po_hbm.at[wid], i2_v)
    pltpu.sync_copy(w1_hbm.at[pl.ds(base, _TPW)], w1_v)
    pltpu.sync_copy(w2_hbm.at[pl.ds(base, _TPW)], w2_v)

    def fire(c, s):
        sl = pl.ds(c * _CCH, _CCH)
        return (pltpu.async_copy(ys_hbm.at[i1_v.at[sl]], r1[s], sg1[s]),
                pltpu.async_copy(ys_hbm.at[i2_v.at[sl]], r2[s], sg2[s]))

    gd = [None] * _CNB
    wb = [None] * _CNB
    for c in range(min(_CNB - 1, _CNC)):
        gd[c] = fire(c, c)
    for c in range(_CNC):
        s = c % _CNB
        n = c + _CNB - 1
        if n < _CNC:
            sn = n % _CNB
            if wb[sn] is not None:
                wb[sn][0].wait()
                wb[sn][1].wait()
            gd[sn] = fire(n, sn)
        gd[s][0].wait()
        gd[s][1].wait()

        def add_row(r, _):
            v1 = w1_v[c * _CCH + r, pl.ds(0, 16)]
            v2 = w2_v[c * _CCH + r, pl.ds(0, 16)]
            bc = lax.bitcast_convert_type

            def add_vec(jb, _):
                for ju in range(8):
                    sl = pl.ds(jb * 128 + ju * 16, 16)
                    p1v = bc(r1[s][r, sl], jnp.int32)
                    p2v = bc(r2[s][r, sl], jnp.int32)
                    lo = (bc(p1v << 16, jnp.float32) * v1
                          + bc(p2v << 16, jnp.float32) * v2)
                    hi = (bc(p1v, jnp.float32) * v1
                          + bc(p2v, jnp.float32) * v2)
                    r1[s][r, sl] = lo
                    r2[s][r, sl] = hi
                return 0
            lax.fori_loop(0, _DH // 128, add_vec, 0)
            return 0

        lax.fori_loop(0, _CCH, add_row, 0)
        out_rows = pl.ds(base + c * _CCH, _CCH)
        wb[s] = (
            pltpu.async_copy(r1[s], out_hbm.at[out_rows, pl.ds(0, _DH)],
                             sw1[s]),
            pltpu.async_copy(r2[s], out_hbm.at[out_rows, pl.ds(_DH, _DH)],
                             sw2[s]))
    for s in range(_CNB):
        if wb[s] is not None:
            wb[s][0].wait()
            wb[s][1].wait()


def _combine(ys, pe, po, w1, w2):
    mesh = plsc.VectorSubcoreMesh(core_axis_name="c", subcore_axis_name="s")
    run = pl.kernel(
        _combine_body,
        out_type=jax.ShapeDtypeStruct((T, D), jnp.float32),
        mesh=mesh,
        scratch_types=(
            [pltpu.VMEM((_TPW,), jnp.int32)] * 2
            + [pltpu.VMEM((_TPW, 16), jnp.float32)] * 2
            + [pltpu.VMEM((_CCH, _DH), jnp.float32)] * (2 * _CNB)
            + [pltpu.SemaphoreType.DMA] * (4 * _CNB)
        ),
    )
    return run(ys, pe, po, w1, w2)


def kernel(hidden_states, gate_weight, w_gate_proj, w_up_proj, w_down_proj):
    x = hidden_states.reshape(T, D)
    w1, w2, xb, pe, po, be, xbi, act = _router_meta(x, gate_weight)
    xs = _dispatch_scatter(xb, pe, po)
    ys = _grouped_ffn(xs, w_gate_proj, w_up_proj, w_down_proj, be, xbi, act)
    out = _combine(ys, pe, po, w1, w2)
    return out.reshape(hidden_states.shape)


# speedup vs baseline: 1.1756x; 1.0140x over previous
import jax
import jax.numpy as jnp
from jax import lax
from jax.experimental import pallas as pl
from jax.experimental.pallas import tpu as pltpu
from jax.experimental.pallas import tpu_sc as plsc

T = 2048
D = 2048
E = 8
F = 768
K = 2

BM = 512
NB = 15
NP = NB * BM

NC, NS = 2, 16
NW = NC * NS
_DH = D // 2
_TPW = T // NW


_RB = 512
_NBL = T // _RB


def _router_body(x_ref, gw_ref, w1_ref, w2_ref, xb_ref, pe_ref, po_ref,
                 be_ref, xbi_ref, act_ref, cnt_ref, lg_ref):
    b = pl.program_id(0)

    def _top2(logits):
        iota = lax.broadcasted_iota(jnp.int32, logits.shape, 1)
        m1 = jnp.max(logits, axis=1, keepdims=True)
        i1 = jnp.min(jnp.where(logits == m1, iota, E), axis=1)
        oh1 = (iota == i1[:, None]).astype(jnp.float32)
        masked = jnp.where(oh1 > 0, -jnp.inf, logits)
        m2 = jnp.max(masked, axis=1, keepdims=True)
        i2 = jnp.min(jnp.where(masked == m2, iota, E), axis=1)
        oh2 = (iota == i2[:, None]).astype(jnp.float32)
        return m1, m2, oh1, oh2

    @pl.when(b < _NBL)
    def _pass1():
        x = x_ref[...]
        gw = gw_ref[...]
        logits = lax.dot_general(x, gw, (((1,), (1,)), ((), ())),
                                 preferred_element_type=jnp.float32)
        lg_ref[pl.ds(b * _RB, _RB), :] = logits
        m1, m2, oh1, oh2 = _top2(logits)
        w1 = jax.nn.sigmoid(m1 - m2)
        w1_ref[...] = jnp.broadcast_to(w1, (_RB, 16))
        w2_ref[...] = jnp.broadcast_to(1.0 - w1, (_RB, 16))

        def _bf16_bits(v):
            u = lax.bitcast_convert_type(v, jnp.int32)
            return (u + 0x7FFF + ((u >> 16) & 1)) >> 16

        blo = _bf16_bits(x[:, :D // 2]) & 0xFFFF
        bhi = _bf16_bits(x[:, D // 2:])
        xb_ref[...] = blo | (bhi << 16)
        cnt_ref[pl.ds(b, 1), :] = jnp.sum(oh1 + oh2, axis=0,
                                          keepdims=True)

    @pl.when(b >= _NBL)
    def _pass2():
        bb = b - _NBL
        _, _, oh1, oh2 = _top2(lg_ref[pl.ds(bb * _RB, _RB), :])
        rows = cnt_ref[...]
        r_iota = lax.broadcasted_iota(jnp.int32, rows.shape, 0)
        c_base = jnp.sum(jnp.where(r_iota < bb, rows, 0.0),
                         axis=0, keepdims=True)
        totals = jnp.sum(rows, axis=0, keepdims=True)
        tot_i = totals.astype(jnp.int32)
        padded = ((tot_i + BM - 1) // BM) * BM
        padded_f = padded.astype(jnp.float32)
        le_i = lax.broadcasted_iota(jnp.int32, (E, E), 0)
        le_j = lax.broadcasted_iota(jnp.int32, (E, E), 1)
        ltri8 = (le_i < le_j).astype(jnp.float32)
        pad_off = lax.dot_general(padded_f, ltri8,
                                  (((1,), (0,)), ((), ())),
                                  preferred_element_type=jnp.float32)
        tt_i = lax.broadcasted_iota(jnp.int32, (_RB, _RB), 0)
        tt_j = lax.broadcasted_iota(jnp.int32, (_RB, _RB), 1)
        strict = (tt_i > tt_j).astype(jnp.float32)
        p_strict = lax.dot_general(strict, oh1 + oh2,
                                   (((1,), (0,)), ((), ())),
                                   preferred_element_type=jnp.float32)
        m = pad_off + c_base + p_strict
        dest1 = jnp.sum(oh1 * m, axis=1).astype(jnp.int32)
        dest2 = jnp.sum(oh2 * m, axis=1).astype(jnp.int32)
        pe_ref[...] = dest1.reshape(_RB // _TPW, _TPW)
        po_ref[...] = dest2.reshape(_RB // _TPW, _TPW)

        @pl.when(b == 2 * _NBL - 1)
        def _tables():
            pad_end = pad_off + padded_f
            nb = (jnp.sum(padded_f) / BM).astype(jnp.int32)
            bi2 = lax.broadcasted_iota(jnp.int32, (NB, E), 0)
            be_raw = jnp.sum((bi2.astype(jnp.float32) * BM >=
                              pad_end).astype(jnp.int32), axis=1)
            b1 = lax.broadcasted_iota(jnp.int32, (NB,), 0)
            active = b1 < nb
            e_last = jnp.sum(jnp.where(b1 == nb - 1, be_raw, 0))
            be_ref[...] = jnp.where(active, be_raw, e_last).astype(jnp.int32)
            xbi_ref[...] = jnp.where(active, b1, nb - 1).astype(jnp.int32)
            act_ref[...] = active.astype(jnp.int32)


def _router_meta(x, gate_weight):
    wpb = _RB // _TPW
    return pl.pallas_call(
        _router_body,
        grid=(2 * _NBL,),
        in_specs=[
            pl.BlockSpec((_RB, D), lambda b: (jnp.minimum(b, _NBL - 1), 0)),
            pl.BlockSpec((E, D), lambda b: (0, 0)),
        ],
        out_specs=[
            pl.BlockSpec((_RB, 16), lambda b: (jnp.minimum(b, _NBL - 1), 0)),
            pl.BlockSpec((_RB, 16), lambda b: (jnp.minimum(b, _NBL - 1), 0)),
            pl.BlockSpec((_RB, _DH),
                         lambda b: (jnp.minimum(b, _NBL - 1), 0)),
            pl.BlockSpec((wpb, _TPW),
                         lambda b: (jnp.maximum(b - _NBL, 0), 0)),
            pl.BlockSpec((wpb, _TPW),
                         lambda b: (jnp.maximum(b - _NBL, 0), 0)),
            pl.BlockSpec((NB,), lambda b: (0,)),
            pl.BlockSpec((NB,), lambda b: (0,)),
            pl.BlockSpec((NB,), lambda b: (0,)),
        ],
        out_shape=[
            jax.ShapeDtypeStruct((T, 16), jnp.float32),
            jax.ShapeDtypeStruct((T, 16), jnp.float32),
            jax.ShapeDtypeStruct((T, _DH), jnp.int32),
            jax.ShapeDtypeStruct((NW, _TPW), jnp.int32),
            jax.ShapeDtypeStruct((NW, _TPW), jnp.int32),
            jax.ShapeDtypeStruct((NB,), jnp.int32),
            jax.ShapeDtypeStruct((NB,), jnp.int32),
            jax.ShapeDtypeStruct((NB,), jnp.int32),
        ],
        scratch_shapes=[pltpu.VMEM((_NBL, E), jnp.float32),
                        pltpu.VMEM((T, E), jnp.float32)],
        compiler_params=pltpu.CompilerParams(
            dimension_semantics=("arbitrary",)),
    )(x, gate_weight)


def _scatter_body(xb_hbm, pe_hbm, po_hbm, xs_hbm,
                  ie_v, io_v, rows_v, s1, s2):
    wid = lax.axis_index("s") * NC + lax.axis_index("c")
    tb = wid * _TPW
    p1 = pltpu.async_copy(pe_hbm.at[wid], ie_v, s1)
    p2 = pltpu.async_copy(po_hbm.at[wid], io_v, s2)
    pltpu.sync_copy(xb_hbm.at[pl.ds(tb, _TPW)], rows_v)
    p1.wait()
    p2.wait()
    c1 = pltpu.async_copy(rows_v, xs_hbm.at[ie_v], s1)
    c2 = pltpu.async_copy(rows_v, xs_hbm.at[io_v], s2)
    c1.wait()
    c2.wait()


def _dispatch_scatter(xb, pe, po):
    mesh = plsc.VectorSubcoreMesh(core_axis_name="c", subcore_axis_name="s")
    run = pl.kernel(
        _scatter_body,
        out_type=jax.ShapeDtypeStruct((NP, _DH), jnp.int32),
        mesh=mesh,
        scratch_types=[
            pltpu.VMEM((_TPW,), jnp.int32),
            pltpu.VMEM((_TPW,), jnp.int32),
            pltpu.VMEM((_TPW, _DH), jnp.int32),
            pltpu.SemaphoreType.DMA,
            pltpu.SemaphoreType.DMA,
        ],
    )
    return run(xb, pe, po)


def _ffn_body(be_ref, xbi_ref, act_ref, xs_ref,
              wg_ref, wu_ref, wd_ref, ys_ref):
    b = pl.program_id(0)

    @pl.when(act_ref[b] == 1)
    def _():
        packed = xs_ref[...]
        xlo = lax.bitcast_convert_type(packed << 16, jnp.float32)
        xhi = lax.bitcast_convert_type(packed & jnp.int32(-65536),
                                       jnp.float32)
        xb = jnp.concatenate([xlo, xhi], axis=1).astype(jnp.bfloat16)
        wg = wg_ref[0].astype(jnp.bfloat16)
        wu = wu_ref[0].astype(jnp.bfloat16)
        wd = wd_ref[0].astype(jnp.bfloat16)
        g = jnp.dot(xb, wg, preferred_element_type=jnp.float32)
        u = jnp.dot(xb, wu, preferred_element_type=jnp.float32)
        h = (g * jax.nn.sigmoid(g)) * u
        y = jnp.dot(h.astype(jnp.bfloat16), wd,
                    preferred_element_type=jnp.float32)

        def _bf16_bits(v):
            u32 = lax.bitcast_convert_type(v, jnp.int32)
            return (u32 + 0x7FFF + ((u32 >> 16) & 1)) >> 16

        blo = _bf16_bits(y[:, :D // 2]) & 0xFFFF
        bhi = _bf16_bits(y[:, D // 2:])
        ys_ref[...] = lax.bitcast_convert_type(blo | (bhi << 16),
                                               jnp.float32)


def _grouped_ffn(xs, w_gate, w_up, w_down, be, xbi, act):
    grid_spec = pltpu.PrefetchScalarGridSpec(
        num_scalar_prefetch=3,
        grid=(NB,),
        in_specs=[
            pl.BlockSpec((BM, _DH), lambda b, be, xbi, act: (xbi[b], 0)),
            pl.BlockSpec((1, D, F), lambda b, be, xbi, act: (be[b], 0, 0)),
            pl.BlockSpec((1, D, F), lambda b, be, xbi, act: (be[b], 0, 0)),
            pl.BlockSpec((1, F, D), lambda b, be, xbi, act: (be[b], 0, 0)),
        ],
        out_specs=pl.BlockSpec((BM, _DH), lambda b, be, xbi, act: (xbi[b], 0)),
    )
    return pl.pallas_call(
        _ffn_body,
        grid_spec=grid_spec,
        out_shape=jax.ShapeDtypeStruct((NP, _DH), jnp.float32),
        compiler_params=pltpu.CompilerParams(
            dimension_semantics=("arbitrary",)),
    )(be, xbi, act, xs, w_gate, w_up, w_down)


_CCH = 8
_CNC = _TPW // _CCH
_CNB = 6


def _combine_body(ys_hbm, pe_hbm, po_hbm, w1_hbm, w2_hbm, out_hbm,
                  i1_v, i2_v, w1_v, w2_v, *scr):
    r1 = scr[:_CNB]
    r2 = scr[_CNB:2 * _CNB]
    sg1 = scr[2 * _CNB:3 * _CNB]
    sg2 = scr[3 * _CNB:4 * _CNB]
    sw1 = scr[4 * _CNB:5 * _CNB]
    sw2 = scr[5 * _CNB:]
    wid = lax.axis_index("s") * NC + lax.axis_index("c")
    base = wid * _TPW
    q1 = pltpu.async_copy(pe_hbm.at[wid], i1_v, sg1[0])
    q2 = pltpu.async_copy(po_hbm.at[wid], i2_v, sg2[0])
    q3 = pltpu.async_copy(w1_hbm.at[pl.ds(base, _TPW)], w1_v, sw1[0])
    q4 = pltpu.async_copy(w2_hbm.at[pl.ds(base, _TPW)], w2_v, sw2[0])
    q1.wait()
    q2.wait()
    q3.wait()
    q4.wait()

    def fire(c, s):
        sl = pl.ds(c * _CCH, _CCH)
        return (pltpu.async_copy(ys_hbm.at[i1_v.at[sl]], r1[s], sg1[s]),
                pltpu.async_copy(ys_hbm.at[i2_v.at[sl]], r2[s], sg2[s]))

    gd = [None] * _CNB
    wb = [None] * _CNB
    for c in range(min(_CNB - 1, _CNC)):
        gd[c] = fire(c, c)
    for c in range(_CNC):
        s = c % _CNB
        n = c + _CNB - 1
        if n < _CNC:
            sn = n % _CNB
            if wb[sn] is not None:
                wb[sn][0].wait()
                wb[sn][1].wait()
            gd[sn] = fire(n, sn)
        gd[s][0].wait()
        gd[s][1].wait()

        def add_row(r, _):
            v1 = w1_v[c * _CCH + r, pl.ds(0, 16)]
            v2 = w2_v[c * _CCH + r, pl.ds(0, 16)]
            bc = lax.bitcast_convert_type

            def add_vec(jb, _):
                for ju in range(8):
                    sl = pl.ds(jb * 128 + ju * 16, 16)
                    p1v = bc(r1[s][r, sl], jnp.int32)
                    p2v = bc(r2[s][r, sl], jnp.int32)
                    lo = (bc(p1v << 16, jnp.float32) * v1
                          + bc(p2v << 16, jnp.float32) * v2)
                    hi = (bc(p1v, jnp.float32) * v1
                          + bc(p2v, jnp.float32) * v2)
                    r1[s][r, sl] = lo
                    r2[s][r, sl] = hi
                return 0
            lax.fori_loop(0, _DH // 128, add_vec, 0)
            return 0

        lax.fori_loop(0, _CCH, add_row, 0)
        out_rows = pl.ds(base + c * _CCH, _CCH)
        wb[s] = (
            pltpu.async_copy(r1[s], out_hbm.at[out_rows, pl.ds(0, _DH)],
                             sw1[s]),
            pltpu.async_copy(r2[s], out_hbm.at[out_rows, pl.ds(_DH, _DH)],
                             sw2[s]))
    for s in range(_CNB):
        if wb[s] is not None:
            wb[s][0].wait()
            wb[s][1].wait()


def _combine(ys, pe, po, w1, w2):
    mesh = plsc.VectorSubcoreMesh(core_axis_name="c", subcore_axis_name="s")
    run = pl.kernel(
        _combine_body,
        out_type=jax.ShapeDtypeStruct((T, D), jnp.float32),
        mesh=mesh,
        scratch_types=(
            [pltpu.VMEM((_TPW,), jnp.int32)] * 2
            + [pltpu.VMEM((_TPW, 16), jnp.float32)] * 2
            + [pltpu.VMEM((_CCH, _DH), jnp.float32)] * (2 * _CNB)
            + [pltpu.SemaphoreType.DMA] * (4 * _CNB)
        ),
    )
    return run(ys, pe, po, w1, w2)


def kernel(hidden_states, gate_weight, w_gate_proj, w_up_proj, w_down_proj):
    x = hidden_states.reshape(T, D)
    w1, w2, xb, pe, po, be, xbi, act = _router_meta(x, gate_weight)
    xs = _dispatch_scatter(xb, pe, po)
    ys = _grouped_ffn(xs, w_gate_proj, w_up_proj, w_down_proj, be, xbi, act)
    out = _combine(ys, pe, po, w1, w2)
    return out.reshape(hidden_states.shape)


# speedup vs baseline: 1.1789x; 1.0028x over previous
import jax
import jax.numpy as jnp
from jax import lax
from jax.experimental import pallas as pl
from jax.experimental.pallas import tpu as pltpu
from jax.experimental.pallas import tpu_sc as plsc

T = 2048
D = 2048
E = 8
F = 768
K = 2

BM = 512
NB = 15
NP = NB * BM

NC, NS = 2, 16
NW = NC * NS
_DH = D // 2
_TPW = T // NW


_RB = 512
_NBL = T // _RB


def _router_body(x_ref, gw_ref, w1_ref, w2_ref, xb_ref, pe_ref, po_ref,
                 be_ref, xbi_ref, act_ref, cnt_ref, lg_ref):
    b = pl.program_id(0)

    def _top2(logits):
        iota = lax.broadcasted_iota(jnp.int32, logits.shape, 1)
        m1 = jnp.max(logits, axis=1, keepdims=True)
        i1 = jnp.min(jnp.where(logits == m1, iota, E), axis=1)
        oh1 = (iota == i1[:, None]).astype(jnp.float32)
        masked = jnp.where(oh1 > 0, -jnp.inf, logits)
        m2 = jnp.max(masked, axis=1, keepdims=True)
        i2 = jnp.min(jnp.where(masked == m2, iota, E), axis=1)
        oh2 = (iota == i2[:, None]).astype(jnp.float32)
        return m1, m2, oh1, oh2

    @pl.when(b < _NBL)
    def _pass1():
        x = x_ref[...]
        gw = gw_ref[...]
        logits = lax.dot_general(x, gw, (((1,), (1,)), ((), ())),
                                 preferred_element_type=jnp.float32)
        lg_ref[pl.ds(b * _RB, _RB), :] = logits
        m1, m2, oh1, oh2 = _top2(logits)
        w1 = jax.nn.sigmoid(m1 - m2)
        w1_ref[...] = jnp.broadcast_to(w1, (_RB, 16))
        w2_ref[...] = jnp.broadcast_to(1.0 - w1, (_RB, 16))

        def _bf16_bits(v):
            u = lax.bitcast_convert_type(v, jnp.int32)
            return (u + 0x7FFF + ((u >> 16) & 1)) >> 16

        blo = _bf16_bits(x[:, :D // 2]) & 0xFFFF
        bhi = _bf16_bits(x[:, D // 2:])
        xb_ref[...] = blo | (bhi << 16)
        cnt_ref[pl.ds(b, 1), :] = jnp.sum(oh1 + oh2, axis=0,
                                          keepdims=True)

    @pl.when(b >= _NBL)
    def _pass2():
        bb = b - _NBL
        _, _, oh1, oh2 = _top2(lg_ref[pl.ds(bb * _RB, _RB), :])
        rows = cnt_ref[...]
        r_iota = lax.broadcasted_iota(jnp.int32, rows.shape, 0)
        c_base = jnp.sum(jnp.where(r_iota < bb, rows, 0.0),
                         axis=0, keepdims=True)
        totals = jnp.sum(rows, axis=0, keepdims=True)
        tot_i = totals.astype(jnp.int32)
        padded = ((tot_i + BM - 1) // BM) * BM
        padded_f = padded.astype(jnp.float32)
        le_i = lax.broadcasted_iota(jnp.int32, (E, E), 0)
        le_j = lax.broadcasted_iota(jnp.int32, (E, E), 1)
        ltri8 = (le_i < le_j).astype(jnp.float32)
        pad_off = lax.dot_general(padded_f, ltri8,
                                  (((1,), (0,)), ((), ())),
                                  preferred_element_type=jnp.float32)
        tt_i = lax.broadcasted_iota(jnp.int32, (_RB, _RB), 0)
        tt_j = lax.broadcasted_iota(jnp.int32, (_RB, _RB), 1)
        strict = (tt_i > tt_j).astype(jnp.float32)
        p_strict = lax.dot_general(strict, oh1 + oh2,
                                   (((1,), (0,)), ((), ())),
                                   preferred_element_type=jnp.float32)
        m = pad_off + c_base + p_strict
        dest1 = jnp.sum(oh1 * m, axis=1).astype(jnp.int32)
        dest2 = jnp.sum(oh2 * m, axis=1).astype(jnp.int32)
        pe_ref[...] = dest1.reshape(_RB // _TPW, _TPW)
        po_ref[...] = dest2.reshape(_RB // _TPW, _TPW)

        @pl.when(b == 2 * _NBL - 1)
        def _tables():
            pad_end = pad_off + padded_f
            nb = (jnp.sum(padded_f) / BM).astype(jnp.int32)
            bi2 = lax.broadcasted_iota(jnp.int32, (NB, E), 0)
            be_raw = jnp.sum((bi2.astype(jnp.float32) * BM >=
                              pad_end).astype(jnp.int32), axis=1)
            b1 = lax.broadcasted_iota(jnp.int32, (NB,), 0)
            active = b1 < nb
            e_last = jnp.sum(jnp.where(b1 == nb - 1, be_raw, 0))
            be_ref[...] = jnp.where(active, be_raw, e_last).astype(jnp.int32)
            xbi_ref[...] = jnp.where(active, b1, nb - 1).astype(jnp.int32)
            act_ref[...] = active.astype(jnp.int32)


def _router_meta(x, gate_weight):
    wpb = _RB // _TPW
    return pl.pallas_call(
        _router_body,
        grid=(2 * _NBL,),
        in_specs=[
            pl.BlockSpec((_RB, D), lambda b: (jnp.minimum(b, _NBL - 1), 0)),
            pl.BlockSpec((E, D), lambda b: (0, 0)),
        ],
        out_specs=[
            pl.BlockSpec((_RB, 16), lambda b: (jnp.minimum(b, _NBL - 1), 0)),
            pl.BlockSpec((_RB, 16), lambda b: (jnp.minimum(b, _NBL - 1), 0)),
            pl.BlockSpec((_RB, _DH),
                         lambda b: (jnp.minimum(b, _NBL - 1), 0)),
            pl.BlockSpec((wpb, _TPW),
                         lambda b: (jnp.maximum(b - _NBL, 0), 0)),
            pl.BlockSpec((wpb, _TPW),
                         lambda b: (jnp.maximum(b - _NBL, 0), 0)),
            pl.BlockSpec((NB,), lambda b: (0,)),
            pl.BlockSpec((NB,), lambda b: (0,)),
            pl.BlockSpec((NB,), lambda b: (0,)),
        ],
        out_shape=[
            jax.ShapeDtypeStruct((T, 16), jnp.float32),
            jax.ShapeDtypeStruct((T, 16), jnp.float32),
            jax.ShapeDtypeStruct((T, _DH), jnp.int32),
            jax.ShapeDtypeStruct((NW, _TPW), jnp.int32),
            jax.ShapeDtypeStruct((NW, _TPW), jnp.int32),
            jax.ShapeDtypeStruct((NB,), jnp.int32),
            jax.ShapeDtypeStruct((NB,), jnp.int32),
            jax.ShapeDtypeStruct((NB,), jnp.int32),
        ],
        scratch_shapes=[pltpu.VMEM((_NBL, E), jnp.float32),
                        pltpu.VMEM((T, E), jnp.float32)],
        compiler_params=pltpu.CompilerParams(
            dimension_semantics=("arbitrary",)),
    )(x, gate_weight)


def _scatter_body(xb_hbm, pe_hbm, po_hbm, xs_hbm,
                  ie_v, io_v, rows_v, s1, s2):
    wid = lax.axis_index("s") * NC + lax.axis_index("c")
    tb = wid * _TPW
    p1 = pltpu.async_copy(pe_hbm.at[wid], ie_v, s1)
    p2 = pltpu.async_copy(po_hbm.at[wid], io_v, s2)
    pltpu.sync_copy(xb_hbm.at[pl.ds(tb, _TPW)], rows_v)
    p1.wait()
    p2.wait()
    c1 = pltpu.async_copy(rows_v, xs_hbm.at[ie_v], s1)
    c2 = pltpu.async_copy(rows_v, xs_hbm.at[io_v], s2)
    c1.wait()
    c2.wait()


def _dispatch_scatter(xb, pe, po):
    mesh = plsc.VectorSubcoreMesh(core_axis_name="c", subcore_axis_name="s")
    run = pl.kernel(
        _scatter_body,
        out_type=jax.ShapeDtypeStruct((NP, _DH), jnp.int32),
        mesh=mesh,
        scratch_types=[
            pltpu.VMEM((_TPW,), jnp.int32),
            pltpu.VMEM((_TPW,), jnp.int32),
            pltpu.VMEM((_TPW, _DH), jnp.int32),
            pltpu.SemaphoreType.DMA,
            pltpu.SemaphoreType.DMA,
        ],
    )
    return run(xb, pe, po)


def _ffn_body(be_ref, xbi_ref, act_ref, xs_ref,
              wg_ref, wu_ref, wd_ref, ys_ref):
    b = pl.program_id(0)

    @pl.when(act_ref[b] == 1)
    def _():
        packed = xs_ref[...]
        xlo = lax.bitcast_convert_type(packed << 16, jnp.float32)
        xhi = lax.bitcast_convert_type(packed & jnp.int32(-65536),
                                       jnp.float32)
        xb = jnp.concatenate([xlo, xhi], axis=1).astype(jnp.bfloat16)
        wg = wg_ref[0].astype(jnp.bfloat16)
        wu = wu_ref[0].astype(jnp.bfloat16)
        wd = wd_ref[0].astype(jnp.bfloat16)
        g = jnp.dot(xb, wg, preferred_element_type=jnp.float32)
        u = jnp.dot(xb, wu, preferred_element_type=jnp.float32)
        h = (g * jax.nn.sigmoid(g)) * u
        y = jnp.dot(h.astype(jnp.bfloat16), wd,
                    preferred_element_type=jnp.float32)

        def _bf16_bits(v):
            u32 = lax.bitcast_convert_type(v, jnp.int32)
            return (u32 + 0x7FFF + ((u32 >> 16) & 1)) >> 16

        blo = _bf16_bits(y[:, :D // 2]) & 0xFFFF
        bhi = _bf16_bits(y[:, D // 2:])
        ys_ref[...] = lax.bitcast_convert_type(blo | (bhi << 16),
                                               jnp.float32)


def _grouped_ffn(xs, w_gate, w_up, w_down, be, xbi, act):
    grid_spec = pltpu.PrefetchScalarGridSpec(
        num_scalar_prefetch=3,
        grid=(NB,),
        in_specs=[
            pl.BlockSpec((BM, _DH), lambda b, be, xbi, act: (xbi[b], 0)),
            pl.BlockSpec((1, D, F), lambda b, be, xbi, act: (be[b], 0, 0)),
            pl.BlockSpec((1, D, F), lambda b, be, xbi, act: (be[b], 0, 0)),
            pl.BlockSpec((1, F, D), lambda b, be, xbi, act: (be[b], 0, 0)),
        ],
        out_specs=pl.BlockSpec((BM, _DH), lambda b, be, xbi, act: (xbi[b], 0)),
    )
    return pl.pallas_call(
        _ffn_body,
        grid_spec=grid_spec,
        out_shape=jax.ShapeDtypeStruct((NP, _DH), jnp.float32),
        compiler_params=pltpu.CompilerParams(
            dimension_semantics=("arbitrary",)),
    )(be, xbi, act, xs, w_gate, w_up, w_down)


_CCH = 8
_CNC = _TPW // _CCH
_CNB = 6


def _combine_body(ys_hbm, pe_hbm, po_hbm, w1_hbm, w2_hbm, out_hbm,
                  i1_v, i2_v, w1_v, w2_v, *scr):
    r1 = scr[:_CNB]
    r2 = scr[_CNB:2 * _CNB]
    sg1 = scr[2 * _CNB:3 * _CNB]
    sg2 = scr[3 * _CNB:4 * _CNB]
    sw1 = scr[4 * _CNB:5 * _CNB]
    sw2 = scr[5 * _CNB:]
    wid = lax.axis_index("s") * NC + lax.axis_index("c")
    base = wid * _TPW
    q1 = pltpu.async_copy(pe_hbm.at[wid], i1_v, sg1[0])
    q2 = pltpu.async_copy(po_hbm.at[wid], i2_v, sg2[0])
    q3 = pltpu.async_copy(w1_hbm.at[pl.ds(base, _TPW)], w1_v, sw1[0])
    q4 = pltpu.async_copy(w2_hbm.at[pl.ds(base, _TPW)], w2_v, sw2[0])
    q1.wait()
    q2.wait()

    def fire(c, s):
        sl = pl.ds(c * _CCH, _CCH)
        return (pltpu.async_copy(ys_hbm.at[i1_v.at[sl]], r1[s], sg1[s]),
                pltpu.async_copy(ys_hbm.at[i2_v.at[sl]], r2[s], sg2[s]))

    gd = [None] * _CNB
    wb = [None] * _CNB
    for c in range(min(_CNB - 1, _CNC)):
        gd[c] = fire(c, c)
    q3.wait()
    q4.wait()
    for c in range(_CNC):
        s = c % _CNB
        n = c + _CNB - 1
        if n < _CNC:
            sn = n % _CNB
            if wb[sn] is not None:
                wb[sn][0].wait()
                wb[sn][1].wait()
            gd[sn] = fire(n, sn)
        gd[s][0].wait()
        gd[s][1].wait()

        def add_row(r, _):
            v1 = w1_v[c * _CCH + r, pl.ds(0, 16)]
            v2 = w2_v[c * _CCH + r, pl.ds(0, 16)]
            bc = lax.bitcast_convert_type

            def add_vec(jb, _):
                for ju in range(8):
                    sl = pl.ds(jb * 128 + ju * 16, 16)
                    p1v = bc(r1[s][r, sl], jnp.int32)
                    p2v = bc(r2[s][r, sl], jnp.int32)
                    lo = (bc(p1v << 16, jnp.float32) * v1
                          + bc(p2v << 16, jnp.float32) * v2)
                    hi = (bc(p1v, jnp.float32) * v1
                          + bc(p2v, jnp.float32) * v2)
                    r1[s][r, sl] = lo
                    r2[s][r, sl] = hi
                return 0
            lax.fori_loop(0, _DH // 128, add_vec, 0)
            return 0

        lax.fori_loop(0, _CCH, add_row, 0)
        out_rows = pl.ds(base + c * _CCH, _CCH)
        wb[s] = (
            pltpu.async_copy(r1[s], out_hbm.at[out_rows, pl.ds(0, _DH)],
                             sw1[s]),
            pltpu.async_copy(r2[s], out_hbm.at[out_rows, pl.ds(_DH, _DH)],
                             sw2[s]))
    for s in range(_CNB):
        if wb[s] is not None:
            wb[s][0].wait()
            wb[s][1].wait()


def _combine(ys, pe, po, w1, w2):
    mesh = plsc.VectorSubcoreMesh(core_axis_name="c", subcore_axis_name="s")
    run = pl.kernel(
        _combine_body,
        out_type=jax.ShapeDtypeStruct((T, D), jnp.float32),
        mesh=mesh,
        scratch_types=(
            [pltpu.VMEM((_TPW,), jnp.int32)] * 2
            + [pltpu.VMEM((_TPW, 16), jnp.float32)] * 2
            + [pltpu.VMEM((_CCH, _DH), jnp.float32)] * (2 * _CNB)
            + [pltpu.SemaphoreType.DMA] * (4 * _CNB)
        ),
    )
    return run(ys, pe, po, w1, w2)


def kernel(hidden_states, gate_weight, w_gate_proj, w_up_proj, w_down_proj):
    x = hidden_states.reshape(T, D)
    w1, w2, xb, pe, po, be, xbi, act = _router_meta(x, gate_weight)
    xs = _dispatch_scatter(xb, pe, po)
    ys = _grouped_ffn(xs, w_gate_proj, w_up_proj, w_down_proj, be, xbi, act)
    out = _combine(ys, pe, po, w1, w2)
    return out.reshape(hidden_states.shape)


# speedup vs baseline: 1.3013x; 1.1038x over previous
import jax
import jax.numpy as jnp
from jax import lax
from jax.experimental import pallas as pl
from jax.experimental.pallas import tpu as pltpu
from jax.experimental.pallas import tpu_sc as plsc

T = 2048
D = 2048
E = 8
F = 768
K = 2

BM = 768
NB = 13
NP = NB * BM

NC, NS = 2, 16
NW = NC * NS
_DH = D // 2
_TPW = T // NW


_RB = 512
_NBL = T // _RB


def _router_body(x_ref, gw_ref, w1_ref, w2_ref, xb_ref, pe_ref, po_ref,
                 be_ref, xbi_ref, act_ref, cnt_ref, lg_ref):
    b = pl.program_id(0)

    def _top2(logits):
        iota = lax.broadcasted_iota(jnp.int32, logits.shape, 1)
        m1 = jnp.max(logits, axis=1, keepdims=True)
        i1 = jnp.min(jnp.where(logits == m1, iota, E), axis=1)
        oh1 = (iota == i1[:, None]).astype(jnp.float32)
        masked = jnp.where(oh1 > 0, -jnp.inf, logits)
        m2 = jnp.max(masked, axis=1, keepdims=True)
        i2 = jnp.min(jnp.where(masked == m2, iota, E), axis=1)
        oh2 = (iota == i2[:, None]).astype(jnp.float32)
        return m1, m2, oh1, oh2

    @pl.when(b < _NBL)
    def _pass1():
        x = x_ref[...]
        gw = gw_ref[...]
        logits = lax.dot_general(x, gw, (((1,), (1,)), ((), ())),
                                 preferred_element_type=jnp.float32)
        lg_ref[pl.ds(b * _RB, _RB), :] = logits
        m1, m2, oh1, oh2 = _top2(logits)
        w1 = jax.nn.sigmoid(m1 - m2)
        w1_ref[...] = jnp.broadcast_to(w1, (_RB, 16))
        w2_ref[...] = jnp.broadcast_to(1.0 - w1, (_RB, 16))

        def _bf16_bits(v):
            u = lax.bitcast_convert_type(v, jnp.int32)
            return (u + 0x7FFF + ((u >> 16) & 1)) >> 16

        blo = _bf16_bits(x[:, :D // 2]) & 0xFFFF
        bhi = _bf16_bits(x[:, D // 2:])
        xb_ref[...] = blo | (bhi << 16)
        cnt_ref[pl.ds(b, 1), :] = jnp.sum(oh1 + oh2, axis=0,
                                          keepdims=True)

    @pl.when(b >= _NBL)
    def _pass2():
        bb = b - _NBL
        _, _, oh1, oh2 = _top2(lg_ref[pl.ds(bb * _RB, _RB), :])
        rows = cnt_ref[...]
        r_iota = lax.broadcasted_iota(jnp.int32, rows.shape, 0)
        c_base = jnp.sum(jnp.where(r_iota < bb, rows, 0.0),
                         axis=0, keepdims=True)
        totals = jnp.sum(rows, axis=0, keepdims=True)
        tot_i = totals.astype(jnp.int32)
        padded = ((tot_i + BM - 1) // BM) * BM
        padded_f = padded.astype(jnp.float32)
        le_i = lax.broadcasted_iota(jnp.int32, (E, E), 0)
        le_j = lax.broadcasted_iota(jnp.int32, (E, E), 1)
        ltri8 = (le_i < le_j).astype(jnp.float32)
        pad_off = lax.dot_general(padded_f, ltri8,
                                  (((1,), (0,)), ((), ())),
                                  preferred_element_type=jnp.float32)
        tt_i = lax.broadcasted_iota(jnp.int32, (_RB, _RB), 0)
        tt_j = lax.broadcasted_iota(jnp.int32, (_RB, _RB), 1)
        strict = (tt_i > tt_j).astype(jnp.float32)
        p_strict = lax.dot_general(strict, oh1 + oh2,
                                   (((1,), (0,)), ((), ())),
                                   preferred_element_type=jnp.float32)
        m = pad_off + c_base + p_strict
        dest1 = jnp.sum(oh1 * m, axis=1).astype(jnp.int32)
        dest2 = jnp.sum(oh2 * m, axis=1).astype(jnp.int32)
        pe_ref[...] = dest1.reshape(_RB // _TPW, _TPW)
        po_ref[...] = dest2.reshape(_RB // _TPW, _TPW)

        @pl.when(b == 2 * _NBL - 1)
        def _tables():
            pad_end = pad_off + padded_f
            nb = (jnp.sum(padded_f) / BM).astype(jnp.int32)
            bi2 = lax.broadcasted_iota(jnp.int32, (NB, E), 0)
            be_raw = jnp.sum((bi2.astype(jnp.float32) * BM >=
                              pad_end).astype(jnp.int32), axis=1)
            b1 = lax.broadcasted_iota(jnp.int32, (NB,), 0)
            active = b1 < nb
            e_last = jnp.sum(jnp.where(b1 == nb - 1, be_raw, 0))
            be_ref[...] = jnp.where(active, be_raw, e_last).astype(jnp.int32)
            xbi_ref[...] = jnp.where(active, b1, nb - 1).astype(jnp.int32)
            act_ref[...] = active.astype(jnp.int32)


def _router_meta(x, gate_weight):
    wpb = _RB // _TPW
    return pl.pallas_call(
        _router_body,
        grid=(2 * _NBL,),
        in_specs=[
            pl.BlockSpec((_RB, D), lambda b: (jnp.minimum(b, _NBL - 1), 0)),
            pl.BlockSpec((E, D), lambda b: (0, 0)),
        ],
        out_specs=[
            pl.BlockSpec((_RB, 16), lambda b: (jnp.minimum(b, _NBL - 1), 0)),
            pl.BlockSpec((_RB, 16), lambda b: (jnp.minimum(b, _NBL - 1), 0)),
            pl.BlockSpec((_RB, _DH),
                         lambda b: (jnp.minimum(b, _NBL - 1), 0)),
            pl.BlockSpec((wpb, _TPW),
                         lambda b: (jnp.maximum(b - _NBL, 0), 0)),
            pl.BlockSpec((wpb, _TPW),
                         lambda b: (jnp.maximum(b - _NBL, 0), 0)),
            pl.BlockSpec((NB,), lambda b: (0,)),
            pl.BlockSpec((NB,), lambda b: (0,)),
            pl.BlockSpec((NB,), lambda b: (0,)),
        ],
        out_shape=[
            jax.ShapeDtypeStruct((T, 16), jnp.float32),
            jax.ShapeDtypeStruct((T, 16), jnp.float32),
            jax.ShapeDtypeStruct((T, _DH), jnp.int32),
            jax.ShapeDtypeStruct((NW, _TPW), jnp.int32),
            jax.ShapeDtypeStruct((NW, _TPW), jnp.int32),
            jax.ShapeDtypeStruct((NB,), jnp.int32),
            jax.ShapeDtypeStruct((NB,), jnp.int32),
            jax.ShapeDtypeStruct((NB,), jnp.int32),
        ],
        scratch_shapes=[pltpu.VMEM((_NBL, E), jnp.float32),
                        pltpu.VMEM((T, E), jnp.float32)],
        compiler_params=pltpu.CompilerParams(
            dimension_semantics=("arbitrary",)),
    )(x, gate_weight)


def _scatter_body(xb_hbm, pe_hbm, po_hbm, xs_hbm,
                  ie_v, io_v, rows_v, s1, s2):
    wid = lax.axis_index("s") * NC + lax.axis_index("c")
    tb = wid * _TPW
    p1 = pltpu.async_copy(pe_hbm.at[wid], ie_v, s1)
    p2 = pltpu.async_copy(po_hbm.at[wid], io_v, s2)
    pltpu.sync_copy(xb_hbm.at[pl.ds(tb, _TPW)], rows_v)
    p1.wait()
    p2.wait()
    c1 = pltpu.async_copy(rows_v, xs_hbm.at[ie_v], s1)
    c2 = pltpu.async_copy(rows_v, xs_hbm.at[io_v], s2)
    c1.wait()
    c2.wait()


def _dispatch_scatter(xb, pe, po):
    mesh = plsc.VectorSubcoreMesh(core_axis_name="c", subcore_axis_name="s")
    run = pl.kernel(
        _scatter_body,
        out_type=jax.ShapeDtypeStruct((NP, _DH), jnp.int32),
        mesh=mesh,
        scratch_types=[
            pltpu.VMEM((_TPW,), jnp.int32),
            pltpu.VMEM((_TPW,), jnp.int32),
            pltpu.VMEM((_TPW, _DH), jnp.int32),
            pltpu.SemaphoreType.DMA,
            pltpu.SemaphoreType.DMA,
        ],
    )
    return run(xb, pe, po)


def _ffn_body(be_ref, xbi_ref, act_ref, xs_ref,
              wg_ref, wu_ref, wd_ref, ys_ref):
    b = pl.program_id(0)

    @pl.when(act_ref[b] == 1)
    def _():
        packed = xs_ref[...]
        xlo = lax.bitcast_convert_type(packed << 16, jnp.float32)
        xhi = lax.bitcast_convert_type(packed & jnp.int32(-65536),
                                       jnp.float32)
        xb = jnp.concatenate([xlo, xhi], axis=1).astype(jnp.bfloat16)
        wg = wg_ref[0].astype(jnp.bfloat16)
        wu = wu_ref[0].astype(jnp.bfloat16)
        wd = wd_ref[0].astype(jnp.bfloat16)
        g = jnp.dot(xb, wg, preferred_element_type=jnp.float32)
        u = jnp.dot(xb, wu, preferred_element_type=jnp.float32)
        h = (g * jax.nn.sigmoid(g)) * u
        y = jnp.dot(h.astype(jnp.bfloat16), wd,
                    preferred_element_type=jnp.float32)

        def _bf16_bits(v):
            u32 = lax.bitcast_convert_type(v, jnp.int32)
            return (u32 + 0x7FFF + ((u32 >> 16) & 1)) >> 16

        blo = _bf16_bits(y[:, :D // 2]) & 0xFFFF
        bhi = _bf16_bits(y[:, D // 2:])
        ys_ref[...] = lax.bitcast_convert_type(blo | (bhi << 16),
                                               jnp.float32)


def _grouped_ffn(xs, w_gate, w_up, w_down, be, xbi, act):
    grid_spec = pltpu.PrefetchScalarGridSpec(
        num_scalar_prefetch=3,
        grid=(NB,),
        in_specs=[
            pl.BlockSpec((BM, _DH), lambda b, be, xbi, act: (xbi[b], 0)),
            pl.BlockSpec((1, D, F), lambda b, be, xbi, act: (be[b], 0, 0)),
            pl.BlockSpec((1, D, F), lambda b, be, xbi, act: (be[b], 0, 0)),
            pl.BlockSpec((1, F, D), lambda b, be, xbi, act: (be[b], 0, 0)),
        ],
        out_specs=pl.BlockSpec((BM, _DH), lambda b, be, xbi, act: (xbi[b], 0)),
    )
    return pl.pallas_call(
        _ffn_body,
        grid_spec=grid_spec,
        out_shape=jax.ShapeDtypeStruct((NP, _DH), jnp.float32),
        compiler_params=pltpu.CompilerParams(
            dimension_semantics=("arbitrary",)),
    )(be, xbi, act, xs, w_gate, w_up, w_down)


_CCH = 8
_CNC = _TPW // _CCH
_CNB = 6


def _combine_body(ys_hbm, pe_hbm, po_hbm, w1_hbm, w2_hbm, out_hbm,
                  i1_v, i2_v, w1_v, w2_v, *scr):
    r1 = scr[:_CNB]
    r2 = scr[_CNB:2 * _CNB]
    sg1 = scr[2 * _CNB:3 * _CNB]
    sg2 = scr[3 * _CNB:4 * _CNB]
    sw1 = scr[4 * _CNB:5 * _CNB]
    sw2 = scr[5 * _CNB:]
    wid = lax.axis_index("s") * NC + lax.axis_index("c")
    base = wid * _TPW
    q1 = pltpu.async_copy(pe_hbm.at[wid], i1_v, sg1[0])
    q2 = pltpu.async_copy(po_hbm.at[wid], i2_v, sg2[0])
    q3 = pltpu.async_copy(w1_hbm.at[pl.ds(base, _TPW)], w1_v, sw1[0])
    q4 = pltpu.async_copy(w2_hbm.at[pl.ds(base, _TPW)], w2_v, sw2[0])
    q1.wait()
    q2.wait()

    def fire(c, s):
        sl = pl.ds(c * _CCH, _CCH)
        return (pltpu.async_copy(ys_hbm.at[i1_v.at[sl]], r1[s], sg1[s]),
                pltpu.async_copy(ys_hbm.at[i2_v.at[sl]], r2[s], sg2[s]))

    gd = [None] * _CNB
    wb = [None] * _CNB
    for c in range(min(_CNB - 1, _CNC)):
        gd[c] = fire(c, c)
    q3.wait()
    q4.wait()
    for c in range(_CNC):
        s = c % _CNB
        n = c + _CNB - 1
        if n < _CNC:
            sn = n % _CNB
            if wb[sn] is not None:
                wb[sn][0].wait()
                wb[sn][1].wait()
            gd[sn] = fire(n, sn)
        gd[s][0].wait()
        gd[s][1].wait()

        def add_row(r, _):
            v1 = w1_v[c * _CCH + r, pl.ds(0, 16)]
            v2 = w2_v[c * _CCH + r, pl.ds(0, 16)]
            bc = lax.bitcast_convert_type

            def add_vec(jb, _):
                for ju in range(8):
                    sl = pl.ds(jb * 128 + ju * 16, 16)
                    p1v = bc(r1[s][r, sl], jnp.int32)
                    p2v = bc(r2[s][r, sl], jnp.int32)
                    lo = (bc(p1v << 16, jnp.float32) * v1
                          + bc(p2v << 16, jnp.float32) * v2)
                    hi = (bc(p1v, jnp.float32) * v1
                          + bc(p2v, jnp.float32) * v2)
                    r1[s][r, sl] = lo
                    r2[s][r, sl] = hi
                return 0
            lax.fori_loop(0, _DH // 128, add_vec, 0)
            return 0

        lax.fori_loop(0, _CCH, add_row, 0)
        out_rows = pl.ds(base + c * _CCH, _CCH)
        wb[s] = (
            pltpu.async_copy(r1[s], out_hbm.at[out_rows, pl.ds(0, _DH)],
                             sw1[s]),
            pltpu.async_copy(r2[s], out_hbm.at[out_rows, pl.ds(_DH, _DH)],
                             sw2[s]))
    for s in range(_CNB):
        if wb[s] is not None:
            wb[s][0].wait()
            wb[s][1].wait()


def _combine(ys, pe, po, w1, w2):
    mesh = plsc.VectorSubcoreMesh(core_axis_name="c", subcore_axis_name="s")
    run = pl.kernel(
        _combine_body,
        out_type=jax.ShapeDtypeStruct((T, D), jnp.float32),
        mesh=mesh,
        scratch_types=(
            [pltpu.VMEM((_TPW,), jnp.int32)] * 2
            + [pltpu.VMEM((_TPW, 16), jnp.float32)] * 2
            + [pltpu.VMEM((_CCH, _DH), jnp.float32)] * (2 * _CNB)
            + [pltpu.SemaphoreType.DMA] * (4 * _CNB)
        ),
    )
    return run(ys, pe, po, w1, w2)


def kernel(hidden_states, gate_weight, w_gate_proj, w_up_proj, w_down_proj):
    x = hidden_states.reshape(T, D)
    w1, w2, xb, pe, po, be, xbi, act = _router_meta(x, gate_weight)
    xs = _dispatch_scatter(xb, pe, po)
    ys = _grouped_ffn(xs, w_gate_proj, w_up_proj, w_down_proj, be, xbi, act)
    out = _combine(ys, pe, po, w1, w2)
    return out.reshape(hidden_states.shape)


# speedup vs baseline: 1.3798x; 1.0603x over previous
import jax
import jax.numpy as jnp
from jax import lax
from jax.experimental import pallas as pl
from jax.experimental.pallas import tpu as pltpu
from jax.experimental.pallas import tpu_sc as plsc

T = 2048
D = 2048
E = 8
F = 768
K = 2

BM = 640
NB = 14
NP = NB * BM

NC, NS = 2, 16
NW = NC * NS
_DH = D // 2
_TPW = T // NW


_RB = 512
_NBL = T // _RB


def _router_body(x_ref, gw_ref, w1_ref, w2_ref, xb_ref, pe_ref, po_ref,
                 be_ref, xbi_ref, act_ref, cnt_ref, lg_ref):
    b = pl.program_id(0)

    def _top2(logits):
        iota = lax.broadcasted_iota(jnp.int32, logits.shape, 1)
        m1 = jnp.max(logits, axis=1, keepdims=True)
        i1 = jnp.min(jnp.where(logits == m1, iota, E), axis=1)
        oh1 = (iota == i1[:, None]).astype(jnp.float32)
        masked = jnp.where(oh1 > 0, -jnp.inf, logits)
        m2 = jnp.max(masked, axis=1, keepdims=True)
        i2 = jnp.min(jnp.where(masked == m2, iota, E), axis=1)
        oh2 = (iota == i2[:, None]).astype(jnp.float32)
        return m1, m2, oh1, oh2

    @pl.when(b < _NBL)
    def _pass1():
        x = x_ref[...]
        gw = gw_ref[...]
        logits = lax.dot_general(x, gw, (((1,), (1,)), ((), ())),
                                 preferred_element_type=jnp.float32)
        lg_ref[pl.ds(b * _RB, _RB), :] = logits
        m1, m2, oh1, oh2 = _top2(logits)
        w1 = jax.nn.sigmoid(m1 - m2)
        w1_ref[...] = jnp.broadcast_to(w1, (_RB, 16))
        w2_ref[...] = jnp.broadcast_to(1.0 - w1, (_RB, 16))

        def _bf16_bits(v):
            u = lax.bitcast_convert_type(v, jnp.int32)
            return (u + 0x7FFF + ((u >> 16) & 1)) >> 16

        blo = _bf16_bits(x[:, :D // 2]) & 0xFFFF
        bhi = _bf16_bits(x[:, D // 2:])
        xb_ref[...] = blo | (bhi << 16)
        cnt_ref[pl.ds(b, 1), :] = jnp.sum(oh1 + oh2, axis=0,
                                          keepdims=True)

    @pl.when(b >= _NBL)
    def _pass2():
        bb = b - _NBL
        _, _, oh1, oh2 = _top2(lg_ref[pl.ds(bb * _RB, _RB), :])
        rows = cnt_ref[...]
        r_iota = lax.broadcasted_iota(jnp.int32, rows.shape, 0)
        c_base = jnp.sum(jnp.where(r_iota < bb, rows, 0.0),
                         axis=0, keepdims=True)
        totals = jnp.sum(rows, axis=0, keepdims=True)
        tot_i = totals.astype(jnp.int32)
        padded = ((tot_i + BM - 1) // BM) * BM
        padded_f = padded.astype(jnp.float32)
        le_i = lax.broadcasted_iota(jnp.int32, (E, E), 0)
        le_j = lax.broadcasted_iota(jnp.int32, (E, E), 1)
        ltri8 = (le_i < le_j).astype(jnp.float32)
        pad_off = lax.dot_general(padded_f, ltri8,
                                  (((1,), (0,)), ((), ())),
                                  preferred_element_type=jnp.float32)
        tt_i = lax.broadcasted_iota(jnp.int32, (_RB, _RB), 0)
        tt_j = lax.broadcasted_iota(jnp.int32, (_RB, _RB), 1)
        strict = (tt_i > tt_j).astype(jnp.float32)
        p_strict = lax.dot_general(strict, oh1 + oh2,
                                   (((1,), (0,)), ((), ())),
                                   preferred_element_type=jnp.float32)
        m = pad_off + c_base + p_strict
        dest1 = jnp.sum(oh1 * m, axis=1).astype(jnp.int32)
        dest2 = jnp.sum(oh2 * m, axis=1).astype(jnp.int32)
        pe_ref[...] = dest1.reshape(_RB // _TPW, _TPW)
        po_ref[...] = dest2.reshape(_RB // _TPW, _TPW)

        @pl.when(b == 2 * _NBL - 1)
        def _tables():
            pad_end = pad_off + padded_f
            nb = (jnp.sum(padded_f) / BM).astype(jnp.int32)
            bi2 = lax.broadcasted_iota(jnp.int32, (NB, E), 0)
            be_raw = jnp.sum((bi2.astype(jnp.float32) * BM >=
                              pad_end).astype(jnp.int32), axis=1)
            b1 = lax.broadcasted_iota(jnp.int32, (NB,), 0)
            active = b1 < nb
            e_last = jnp.sum(jnp.where(b1 == nb - 1, be_raw, 0))
            be_ref[...] = jnp.where(active, be_raw, e_last).astype(jnp.int32)
            xbi_ref[...] = jnp.where(active, b1, nb - 1).astype(jnp.int32)
            act_ref[...] = active.astype(jnp.int32)


def _router_meta(x, gate_weight):
    wpb = _RB // _TPW
    return pl.pallas_call(
        _router_body,
        grid=(2 * _NBL,),
        in_specs=[
            pl.BlockSpec((_RB, D), lambda b: (jnp.minimum(b, _NBL - 1), 0)),
            pl.BlockSpec((E, D), lambda b: (0, 0)),
        ],
        out_specs=[
            pl.BlockSpec((_RB, 16), lambda b: (jnp.minimum(b, _NBL - 1), 0)),
            pl.BlockSpec((_RB, 16), lambda b: (jnp.minimum(b, _NBL - 1), 0)),
            pl.BlockSpec((_RB, _DH),
                         lambda b: (jnp.minimum(b, _NBL - 1), 0)),
            pl.BlockSpec((wpb, _TPW),
                         lambda b: (jnp.maximum(b - _NBL, 0), 0)),
            pl.BlockSpec((wpb, _TPW),
                         lambda b: (jnp.maximum(b - _NBL, 0), 0)),
            pl.BlockSpec((NB,), lambda b: (0,)),
            pl.BlockSpec((NB,), lambda b: (0,)),
            pl.BlockSpec((NB,), lambda b: (0,)),
        ],
        out_shape=[
            jax.ShapeDtypeStruct((T, 16), jnp.float32),
            jax.ShapeDtypeStruct((T, 16), jnp.float32),
            jax.ShapeDtypeStruct((T, _DH), jnp.int32),
            jax.ShapeDtypeStruct((NW, _TPW), jnp.int32),
            jax.ShapeDtypeStruct((NW, _TPW), jnp.int32),
            jax.ShapeDtypeStruct((NB,), jnp.int32),
            jax.ShapeDtypeStruct((NB,), jnp.int32),
            jax.ShapeDtypeStruct((NB,), jnp.int32),
        ],
        scratch_shapes=[pltpu.VMEM((_NBL, E), jnp.float32),
                        pltpu.VMEM((T, E), jnp.float32)],
        compiler_params=pltpu.CompilerParams(
            dimension_semantics=("arbitrary",)),
    )(x, gate_weight)


def _scatter_body(xb_hbm, pe_hbm, po_hbm, xs_hbm,
                  ie_v, io_v, rows_v, s1, s2):
    wid = lax.axis_index("s") * NC + lax.axis_index("c")
    tb = wid * _TPW
    p1 = pltpu.async_copy(pe_hbm.at[wid], ie_v, s1)
    p2 = pltpu.async_copy(po_hbm.at[wid], io_v, s2)
    pltpu.sync_copy(xb_hbm.at[pl.ds(tb, _TPW)], rows_v)
    p1.wait()
    p2.wait()
    c1 = pltpu.async_copy(rows_v, xs_hbm.at[ie_v], s1)
    c2 = pltpu.async_copy(rows_v, xs_hbm.at[io_v], s2)
    c1.wait()
    c2.wait()


def _dispatch_scatter(xb, pe, po):
    mesh = plsc.VectorSubcoreMesh(core_axis_name="c", subcore_axis_name="s")
    run = pl.kernel(
        _scatter_body,
        out_type=jax.ShapeDtypeStruct((NP, _DH), jnp.int32),
        mesh=mesh,
        scratch_types=[
            pltpu.VMEM((_TPW,), jnp.int32),
            pltpu.VMEM((_TPW,), jnp.int32),
            pltpu.VMEM((_TPW, _DH), jnp.int32),
            pltpu.SemaphoreType.DMA,
            pltpu.SemaphoreType.DMA,
        ],
    )
    return run(xb, pe, po)


def _ffn_body(be_ref, xbi_ref, act_ref, xs_ref,
              wg_ref, wu_ref, wd_ref, ys_ref):
    b = pl.program_id(0)

    @pl.when(act_ref[b] == 1)
    def _():
        packed = xs_ref[...]
        xlo = lax.bitcast_convert_type(packed << 16, jnp.float32)
        xhi = lax.bitcast_convert_type(packed & jnp.int32(-65536),
                                       jnp.float32)
        xb = jnp.concatenate([xlo, xhi], axis=1).astype(jnp.bfloat16)
        wg = wg_ref[0].astype(jnp.bfloat16)
        wu = wu_ref[0].astype(jnp.bfloat16)
        wd = wd_ref[0].astype(jnp.bfloat16)
        g = jnp.dot(xb, wg, preferred_element_type=jnp.float32)
        u = jnp.dot(xb, wu, preferred_element_type=jnp.float32)
        h = (g * jax.nn.sigmoid(g)) * u
        y = jnp.dot(h.astype(jnp.bfloat16), wd,
                    preferred_element_type=jnp.float32)

        def _bf16_bits(v):
            u32 = lax.bitcast_convert_type(v, jnp.int32)
            return (u32 + 0x7FFF + ((u32 >> 16) & 1)) >> 16

        blo = _bf16_bits(y[:, :D // 2]) & 0xFFFF
        bhi = _bf16_bits(y[:, D // 2:])
        ys_ref[...] = lax.bitcast_convert_type(blo | (bhi << 16),
                                               jnp.float32)


def _grouped_ffn(xs, w_gate, w_up, w_down, be, xbi, act):
    grid_spec = pltpu.PrefetchScalarGridSpec(
        num_scalar_prefetch=3,
        grid=(NB,),
        in_specs=[
            pl.BlockSpec((BM, _DH), lambda b, be, xbi, act: (xbi[b], 0)),
            pl.BlockSpec((1, D, F), lambda b, be, xbi, act: (be[b], 0, 0)),
            pl.BlockSpec((1, D, F), lambda b, be, xbi, act: (be[b], 0, 0)),
            pl.BlockSpec((1, F, D), lambda b, be, xbi, act: (be[b], 0, 0)),
        ],
        out_specs=pl.BlockSpec((BM, _DH), lambda b, be, xbi, act: (xbi[b], 0)),
    )
    return pl.pallas_call(
        _ffn_body,
        grid_spec=grid_spec,
        out_shape=jax.ShapeDtypeStruct((NP, _DH), jnp.float32),
        compiler_params=pltpu.CompilerParams(
            dimension_semantics=("arbitrary",)),
    )(be, xbi, act, xs, w_gate, w_up, w_down)


_CCH = 8
_CNC = _TPW // _CCH
_CNB = 6


def _combine_body(ys_hbm, pe_hbm, po_hbm, w1_hbm, w2_hbm, out_hbm,
                  i1_v, i2_v, w1_v, w2_v, *scr):
    r1 = scr[:_CNB]
    r2 = scr[_CNB:2 * _CNB]
    sg1 = scr[2 * _CNB:3 * _CNB]
    sg2 = scr[3 * _CNB:4 * _CNB]
    sw1 = scr[4 * _CNB:5 * _CNB]
    sw2 = scr[5 * _CNB:]
    wid = lax.axis_index("s") * NC + lax.axis_index("c")
    base = wid * _TPW
    q1 = pltpu.async_copy(pe_hbm.at[wid], i1_v, sg1[0])
    q2 = pltpu.async_copy(po_hbm.at[wid], i2_v, sg2[0])
    q3 = pltpu.async_copy(w1_hbm.at[pl.ds(base, _TPW)], w1_v, sw1[0])
    q4 = pltpu.async_copy(w2_hbm.at[pl.ds(base, _TPW)], w2_v, sw2[0])
    q1.wait()
    q2.wait()

    def fire(c, s):
        sl = pl.ds(c * _CCH, _CCH)
        return (pltpu.async_copy(ys_hbm.at[i1_v.at[sl]], r1[s], sg1[s]),
                pltpu.async_copy(ys_hbm.at[i2_v.at[sl]], r2[s], sg2[s]))

    gd = [None] * _CNB
    wb = [None] * _CNB
    for c in range(min(_CNB - 1, _CNC)):
        gd[c] = fire(c, c)
    q3.wait()
    q4.wait()
    for c in range(_CNC):
        s = c % _CNB
        n = c + _CNB - 1
        if n < _CNC:
            sn = n % _CNB
            if wb[sn] is not None:
                wb[sn][0].wait()
                wb[sn][1].wait()
            gd[sn] = fire(n, sn)
        gd[s][0].wait()
        gd[s][1].wait()

        def add_row(r, _):
            v1 = w1_v[c * _CCH + r, pl.ds(0, 16)]
            v2 = w2_v[c * _CCH + r, pl.ds(0, 16)]
            bc = lax.bitcast_convert_type

            def add_vec(jb, _):
                for ju in range(8):
                    sl = pl.ds(jb * 128 + ju * 16, 16)
                    p1v = bc(r1[s][r, sl], jnp.int32)
                    p2v = bc(r2[s][r, sl], jnp.int32)
                    lo = (bc(p1v << 16, jnp.float32) * v1
                          + bc(p2v << 16, jnp.float32) * v2)
                    hi = (bc(p1v, jnp.float32) * v1
                          + bc(p2v, jnp.float32) * v2)
                    r1[s][r, sl] = lo
                    r2[s][r, sl] = hi
                return 0
            lax.fori_loop(0, _DH // 128, add_vec, 0)
            return 0

        lax.fori_loop(0, _CCH, add_row, 0)
        out_rows = pl.ds(base + c * _CCH, _CCH)
        wb[s] = (
            pltpu.async_copy(r1[s], out_hbm.at[out_rows, pl.ds(0, _DH)],
                             sw1[s]),
            pltpu.async_copy(r2[s], out_hbm.at[out_rows, pl.ds(_DH, _DH)],
                             sw2[s]))
    for s in range(_CNB):
        if wb[s] is not None:
            wb[s][0].wait()
            wb[s][1].wait()


def _combine(ys, pe, po, w1, w2):
    mesh = plsc.VectorSubcoreMesh(core_axis_name="c", subcore_axis_name="s")
    run = pl.kernel(
        _combine_body,
        out_type=jax.ShapeDtypeStruct((T, D), jnp.float32),
        mesh=mesh,
        scratch_types=(
            [pltpu.VMEM((_TPW,), jnp.int32)] * 2
            + [pltpu.VMEM((_TPW, 16), jnp.float32)] * 2
            + [pltpu.VMEM((_CCH, _DH), jnp.float32)] * (2 * _CNB)
            + [pltpu.SemaphoreType.DMA] * (4 * _CNB)
        ),
    )
    return run(ys, pe, po, w1, w2)


def kernel(hidden_states, gate_weight, w_gate_proj, w_up_proj, w_down_proj):
    x = hidden_states.reshape(T, D)
    w1, w2, xb, pe, po, be, xbi, act = _router_meta(x, gate_weight)
    xs = _dispatch_scatter(xb, pe, po)
    ys = _grouped_ffn(xs, w_gate_proj, w_up_proj, w_down_proj, be, xbi, act)
    out = _combine(ys, pe, po, w1, w2)
    return out.reshape(hidden_states.shape)


# speedup vs baseline: 1.4087x; 1.0209x over previous
import jax
import jax.numpy as jnp
from jax import lax
from jax.experimental import pallas as pl
from jax.experimental.pallas import tpu as pltpu
from jax.experimental.pallas import tpu_sc as plsc

T = 2048
D = 2048
E = 8
F = 768
K = 2

BM = 576
NB = 16
NP = NB * BM

NC, NS = 2, 16
NW = NC * NS
_DH = D // 2
_TPW = T // NW


_RB = 512
_NBL = T // _RB


def _router_body(x_ref, gw_ref, w1_ref, w2_ref, xb_ref, pe_ref, po_ref,
                 be_ref, xbi_ref, act_ref, cnt_ref, lg_ref):
    b = pl.program_id(0)

    def _top2(logits):
        iota = lax.broadcasted_iota(jnp.int32, logits.shape, 1)
        m1 = jnp.max(logits, axis=1, keepdims=True)
        i1 = jnp.min(jnp.where(logits == m1, iota, E), axis=1)
        oh1 = (iota == i1[:, None]).astype(jnp.float32)
        masked = jnp.where(oh1 > 0, -jnp.inf, logits)
        m2 = jnp.max(masked, axis=1, keepdims=True)
        i2 = jnp.min(jnp.where(masked == m2, iota, E), axis=1)
        oh2 = (iota == i2[:, None]).astype(jnp.float32)
        return m1, m2, oh1, oh2

    @pl.when(b < _NBL)
    def _pass1():
        x = x_ref[...]
        gw = gw_ref[...]
        logits = lax.dot_general(x, gw, (((1,), (1,)), ((), ())),
                                 preferred_element_type=jnp.float32)
        lg_ref[pl.ds(b * _RB, _RB), :] = logits
        m1, m2, oh1, oh2 = _top2(logits)
        w1 = jax.nn.sigmoid(m1 - m2)
        w1_ref[...] = jnp.broadcast_to(w1, (_RB, 16))
        w2_ref[...] = jnp.broadcast_to(1.0 - w1, (_RB, 16))

        def _bf16_bits(v):
            u = lax.bitcast_convert_type(v, jnp.int32)
            return (u + 0x7FFF + ((u >> 16) & 1)) >> 16

        blo = _bf16_bits(x[:, :D // 2]) & 0xFFFF
        bhi = _bf16_bits(x[:, D // 2:])
        xb_ref[...] = blo | (bhi << 16)
        cnt_ref[pl.ds(b, 1), :] = jnp.sum(oh1 + oh2, axis=0,
                                          keepdims=True)

    @pl.when(b >= _NBL)
    def _pass2():
        bb = b - _NBL
        _, _, oh1, oh2 = _top2(lg_ref[pl.ds(bb * _RB, _RB), :])
        rows = cnt_ref[...]
        r_iota = lax.broadcasted_iota(jnp.int32, rows.shape, 0)
        c_base = jnp.sum(jnp.where(r_iota < bb, rows, 0.0),
                         axis=0, keepdims=True)
        totals = jnp.sum(rows, axis=0, keepdims=True)
        tot_i = totals.astype(jnp.int32)
        padded = ((tot_i + BM - 1) // BM) * BM
        padded_f = padded.astype(jnp.float32)
        le_i = lax.broadcasted_iota(jnp.int32, (E, E), 0)
        le_j = lax.broadcasted_iota(jnp.int32, (E, E), 1)
        ltri8 = (le_i < le_j).astype(jnp.float32)
        pad_off = lax.dot_general(padded_f, ltri8,
                                  (((1,), (0,)), ((), ())),
                                  preferred_element_type=jnp.float32)
        tt_i = lax.broadcasted_iota(jnp.int32, (_RB, _RB), 0)
        tt_j = lax.broadcasted_iota(jnp.int32, (_RB, _RB), 1)
        strict = (tt_i > tt_j).astype(jnp.float32)
        p_strict = lax.dot_general(strict, oh1 + oh2,
                                   (((1,), (0,)), ((), ())),
                                   preferred_element_type=jnp.float32)
        m = pad_off + c_base + p_strict
        dest1 = jnp.sum(oh1 * m, axis=1).astype(jnp.int32)
        dest2 = jnp.sum(oh2 * m, axis=1).astype(jnp.int32)
        pe_ref[...] = dest1.reshape(_RB // _TPW, _TPW)
        po_ref[...] = dest2.reshape(_RB // _TPW, _TPW)

        @pl.when(b == 2 * _NBL - 1)
        def _tables():
            pad_end = pad_off + padded_f
            nb = (jnp.sum(padded_f) / BM).astype(jnp.int32)
            bi2 = lax.broadcasted_iota(jnp.int32, (NB, E), 0)
            be_raw = jnp.sum((bi2.astype(jnp.float32) * BM >=
                              pad_end).astype(jnp.int32), axis=1)
            b1 = lax.broadcasted_iota(jnp.int32, (NB,), 0)
            active = b1 < nb
            e_last = jnp.sum(jnp.where(b1 == nb - 1, be_raw, 0))
            be_ref[...] = jnp.where(active, be_raw, e_last).astype(jnp.int32)
            xbi_ref[...] = jnp.where(active, b1, nb - 1).astype(jnp.int32)
            act_ref[...] = active.astype(jnp.int32)


def _router_meta(x, gate_weight):
    wpb = _RB // _TPW
    return pl.pallas_call(
        _router_body,
        grid=(2 * _NBL,),
        in_specs=[
            pl.BlockSpec((_RB, D), lambda b: (jnp.minimum(b, _NBL - 1), 0)),
            pl.BlockSpec((E, D), lambda b: (0, 0)),
        ],
        out_specs=[
            pl.BlockSpec((_RB, 16), lambda b: (jnp.minimum(b, _NBL - 1), 0)),
            pl.BlockSpec((_RB, 16), lambda b: (jnp.minimum(b, _NBL - 1), 0)),
            pl.BlockSpec((_RB, _DH),
                         lambda b: (jnp.minimum(b, _NBL - 1), 0)),
            pl.BlockSpec((wpb, _TPW),
                         lambda b: (jnp.maximum(b - _NBL, 0), 0)),
            pl.BlockSpec((wpb, _TPW),
                         lambda b: (jnp.maximum(b - _NBL, 0), 0)),
            pl.BlockSpec((NB,), lambda b: (0,)),
            pl.BlockSpec((NB,), lambda b: (0,)),
            pl.BlockSpec((NB,), lambda b: (0,)),
        ],
        out_shape=[
            jax.ShapeDtypeStruct((T, 16), jnp.float32),
            jax.ShapeDtypeStruct((T, 16), jnp.float32),
            jax.ShapeDtypeStruct((T, _DH), jnp.int32),
            jax.ShapeDtypeStruct((NW, _TPW), jnp.int32),
            jax.ShapeDtypeStruct((NW, _TPW), jnp.int32),
            jax.ShapeDtypeStruct((NB,), jnp.int32),
            jax.ShapeDtypeStruct((NB,), jnp.int32),
            jax.ShapeDtypeStruct((NB,), jnp.int32),
        ],
        scratch_shapes=[pltpu.VMEM((_NBL, E), jnp.float32),
                        pltpu.VMEM((T, E), jnp.float32)],
        compiler_params=pltpu.CompilerParams(
            dimension_semantics=("arbitrary",)),
    )(x, gate_weight)


def _scatter_body(xb_hbm, pe_hbm, po_hbm, xs_hbm,
                  ie_v, io_v, rows_v, s1, s2):
    wid = lax.axis_index("s") * NC + lax.axis_index("c")
    tb = wid * _TPW
    p1 = pltpu.async_copy(pe_hbm.at[wid], ie_v, s1)
    p2 = pltpu.async_copy(po_hbm.at[wid], io_v, s2)
    pltpu.sync_copy(xb_hbm.at[pl.ds(tb, _TPW)], rows_v)
    p1.wait()
    p2.wait()
    c1 = pltpu.async_copy(rows_v, xs_hbm.at[ie_v], s1)
    c2 = pltpu.async_copy(rows_v, xs_hbm.at[io_v], s2)
    c1.wait()
    c2.wait()


def _dispatch_scatter(xb, pe, po):
    mesh = plsc.VectorSubcoreMesh(core_axis_name="c", subcore_axis_name="s")
    run = pl.kernel(
        _scatter_body,
        out_type=jax.ShapeDtypeStruct((NP, _DH), jnp.int32),
        mesh=mesh,
        scratch_types=[
            pltpu.VMEM((_TPW,), jnp.int32),
            pltpu.VMEM((_TPW,), jnp.int32),
            pltpu.VMEM((_TPW, _DH), jnp.int32),
            pltpu.SemaphoreType.DMA,
            pltpu.SemaphoreType.DMA,
        ],
    )
    return run(xb, pe, po)


def _ffn_body(be_ref, xbi_ref, act_ref, xs_ref,
              wg_ref, wu_ref, wd_ref, ys_ref):
    b = pl.program_id(0)

    @pl.when(act_ref[b] == 1)
    def _():
        packed = xs_ref[...]
        xlo = lax.bitcast_convert_type(packed << 16, jnp.float32)
        xhi = lax.bitcast_convert_type(packed & jnp.int32(-65536),
                                       jnp.float32)
        xb = jnp.concatenate([xlo, xhi], axis=1).astype(jnp.bfloat16)
        wg = wg_ref[0].astype(jnp.bfloat16)
        wu = wu_ref[0].astype(jnp.bfloat16)
        wd = wd_ref[0].astype(jnp.bfloat16)
        g = jnp.dot(xb, wg, preferred_element_type=jnp.float32)
        u = jnp.dot(xb, wu, preferred_element_type=jnp.float32)
        h = (g * jax.nn.sigmoid(g)) * u
        y = jnp.dot(h.astype(jnp.bfloat16), wd,
                    preferred_element_type=jnp.float32)

        def _bf16_bits(v):
            u32 = lax.bitcast_convert_type(v, jnp.int32)
            return (u32 + 0x7FFF + ((u32 >> 16) & 1)) >> 16

        blo = _bf16_bits(y[:, :D // 2]) & 0xFFFF
        bhi = _bf16_bits(y[:, D // 2:])
        ys_ref[...] = lax.bitcast_convert_type(blo | (bhi << 16),
                                               jnp.float32)


def _grouped_ffn(xs, w_gate, w_up, w_down, be, xbi, act):
    grid_spec = pltpu.PrefetchScalarGridSpec(
        num_scalar_prefetch=3,
        grid=(NB,),
        in_specs=[
            pl.BlockSpec((BM, _DH), lambda b, be, xbi, act: (xbi[b], 0)),
            pl.BlockSpec((1, D, F), lambda b, be, xbi, act: (be[b], 0, 0)),
            pl.BlockSpec((1, D, F), lambda b, be, xbi, act: (be[b], 0, 0)),
            pl.BlockSpec((1, F, D), lambda b, be, xbi, act: (be[b], 0, 0)),
        ],
        out_specs=pl.BlockSpec((BM, _DH), lambda b, be, xbi, act: (xbi[b], 0)),
    )
    return pl.pallas_call(
        _ffn_body,
        grid_spec=grid_spec,
        out_shape=jax.ShapeDtypeStruct((NP, _DH), jnp.float32),
        compiler_params=pltpu.CompilerParams(
            dimension_semantics=("arbitrary",)),
    )(be, xbi, act, xs, w_gate, w_up, w_down)


_CCH = 8
_CNC = _TPW // _CCH
_CNB = 6


def _combine_body(ys_hbm, pe_hbm, po_hbm, w1_hbm, w2_hbm, out_hbm,
                  i1_v, i2_v, w1_v, w2_v, *scr):
    r1 = scr[:_CNB]
    r2 = scr[_CNB:2 * _CNB]
    sg1 = scr[2 * _CNB:3 * _CNB]
    sg2 = scr[3 * _CNB:4 * _CNB]
    sw1 = scr[4 * _CNB:5 * _CNB]
    sw2 = scr[5 * _CNB:]
    wid = lax.axis_index("s") * NC + lax.axis_index("c")
    base = wid * _TPW
    q1 = pltpu.async_copy(pe_hbm.at[wid], i1_v, sg1[0])
    q2 = pltpu.async_copy(po_hbm.at[wid], i2_v, sg2[0])
    q3 = pltpu.async_copy(w1_hbm.at[pl.ds(base, _TPW)], w1_v, sw1[0])
    q4 = pltpu.async_copy(w2_hbm.at[pl.ds(base, _TPW)], w2_v, sw2[0])
    q1.wait()
    q2.wait()

    def fire(c, s):
        sl = pl.ds(c * _CCH, _CCH)
        return (pltpu.async_copy(ys_hbm.at[i1_v.at[sl]], r1[s], sg1[s]),
                pltpu.async_copy(ys_hbm.at[i2_v.at[sl]], r2[s], sg2[s]))

    gd = [None] * _CNB
    wb = [None] * _CNB
    for c in range(min(_CNB - 1, _CNC)):
        gd[c] = fire(c, c)
    q3.wait()
    q4.wait()
    for c in range(_CNC):
        s = c % _CNB
        n = c + _CNB - 1
        if n < _CNC:
            sn = n % _CNB
            if wb[sn] is not None:
                wb[sn][0].wait()
                wb[sn][1].wait()
            gd[sn] = fire(n, sn)
        gd[s][0].wait()
        gd[s][1].wait()

        def add_row(r, _):
            v1 = w1_v[c * _CCH + r, pl.ds(0, 16)]
            v2 = w2_v[c * _CCH + r, pl.ds(0, 16)]
            bc = lax.bitcast_convert_type

            def add_vec(jb, _):
                for ju in range(8):
                    sl = pl.ds(jb * 128 + ju * 16, 16)
                    p1v = bc(r1[s][r, sl], jnp.int32)
                    p2v = bc(r2[s][r, sl], jnp.int32)
                    lo = (bc(p1v << 16, jnp.float32) * v1
                          + bc(p2v << 16, jnp.float32) * v2)
                    hi = (bc(p1v, jnp.float32) * v1
                          + bc(p2v, jnp.float32) * v2)
                    r1[s][r, sl] = lo
                    r2[s][r, sl] = hi
                return 0
            lax.fori_loop(0, _DH // 128, add_vec, 0)
            return 0

        lax.fori_loop(0, _CCH, add_row, 0)
        out_rows = pl.ds(base + c * _CCH, _CCH)
        wb[s] = (
            pltpu.async_copy(r1[s], out_hbm.at[out_rows, pl.ds(0, _DH)],
                             sw1[s]),
            pltpu.async_copy(r2[s], out_hbm.at[out_rows, pl.ds(_DH, _DH)],
                             sw2[s]))
    for s in range(_CNB):
        if wb[s] is not None:
            wb[s][0].wait()
            wb[s][1].wait()


def _combine(ys, pe, po, w1, w2):
    mesh = plsc.VectorSubcoreMesh(core_axis_name="c", subcore_axis_name="s")
    run = pl.kernel(
        _combine_body,
        out_type=jax.ShapeDtypeStruct((T, D), jnp.float32),
        mesh=mesh,
        scratch_types=(
            [pltpu.VMEM((_TPW,), jnp.int32)] * 2
            + [pltpu.VMEM((_TPW, 16), jnp.float32)] * 2
            + [pltpu.VMEM((_CCH, _DH), jnp.float32)] * (2 * _CNB)
            + [pltpu.SemaphoreType.DMA] * (4 * _CNB)
        ),
    )
    return run(ys, pe, po, w1, w2)


def kernel(hidden_states, gate_weight, w_gate_proj, w_up_proj, w_down_proj):
    x = hidden_states.reshape(T, D)
    w1, w2, xb, pe, po, be, xbi, act = _router_meta(x, gate_weight)
    xs = _dispatch_scatter(xb, pe, po)
    ys = _grouped_ffn(xs, w_gate_proj, w_up_proj, w_down_proj, be, xbi, act)
    out = _combine(ys, pe, po, w1, w2)
    return out.reshape(hidden_states.shape)
